# baseline (speedup 1.0000x reference)
import functools

import jax
import jax.numpy as jnp
from jax import lax
from jax.experimental import pallas as pl
from jax.experimental.pallas import tpu as pltpu
from jax.experimental.pallas import tpu_sc as plsc

N = 10000
E = 160000
D = 256
NB = 10
NIDX = 1000
NC, NS = 2, 16
HALF = D // 2
BATCH = 128
EP = 163840
NBATCH = EP // NS // BATCH
BSZ = 64
NBT = EP // NS // BSZ
PHB = NBT // 4
NPAD = 10240
STRIPE = NPAD // NS
IPAD = 1024
IPT = IPAD // NS
MMR = 1000

_mesh = plsc.VectorSubcoreMesh(core_axis_name="c", subcore_axis_name="s")


def _elu(v):
    return jnp.where(v > 0, v, jnp.exp(v) - 1.0)


@functools.partial(
    pl.kernel,
    out_type=jax.ShapeDtypeStruct((NPAD,), jnp.float32),
    mesh=_mesh,
    compiler_params=pltpu.CompilerParams(needs_layout_passes=False),
    scratch_types=[
        pltpu.VMEM((NBATCH, BATCH), jnp.int32),
        pltpu.VMEM((NPAD,), jnp.float32),
        pltpu.VMEM((NS, STRIPE), jnp.float32),
        pltpu.VMEM((STRIPE,), jnp.float32),
        pltpu.VMEM_SHARED((NS, NPAD), jnp.float32),
    ],
)
def _deg(dst_hbm, out_hbm, dst_v, acc_v, red_v, res_v, shared):
    c = lax.axis_index("c")
    s = lax.axis_index("s")
    pltpu.sync_copy(dst_hbm.at[s], dst_v)
    zf = jnp.zeros((16,), jnp.float32)

    def zero(i, carry):
        acc_v[pl.ds(i * 16, 16)] = zf
        return carry

    lax.fori_loop(0, NPAD // 16, zero, 0)
    ones = jnp.full((16,), 1.0, jnp.float32)

    def row(j, carry):
        def sub(k, carry2):
            idx = dst_v[j, pl.ds(k * 16, 16)]
            plsc.addupdate_scatter(acc_v, [idx], ones)
            return carry2

        return lax.fori_loop(0, BATCH // 16, sub, carry)

    lax.fori_loop(0, NBATCH, row, 0)
    pltpu.sync_copy(acc_v, shared.at[s])
    plsc.subcore_barrier()
    pltpu.sync_copy(shared.at[:, pl.ds(s * STRIPE, STRIPE)], red_v)

    def reduce(i, carry):
        t = jnp.full((16,), 1.0, jnp.float32)
        for k in range(NS):
            t = t + red_v[k, pl.ds(i * 16, 16)]
        res_v[pl.ds(i * 16, 16)] = t
        return carry

    lax.fori_loop(0, STRIPE // 16, reduce, 0)

    @pl.when(c == 0)
    def _():
        pltpu.sync_copy(res_v, out_hbm.at[pl.ds(s * STRIPE, STRIPE)])


@functools.partial(
    pl.kernel,
    out_type=jax.ShapeDtypeStruct((NC, N, HALF), jnp.float32),
    mesh=_mesh,
    compiler_params=pltpu.CompilerParams(needs_layout_passes=False),
    scratch_types=[
        pltpu.VMEM((PHB, BSZ), jnp.int32),
        pltpu.VMEM((PHB, BSZ), jnp.int32),
        pltpu.VMEM((BSZ, HALF), jnp.float32),
        pltpu.VMEM((BSZ, HALF), jnp.float32),
        pltpu.VMEM((BSZ, HALF), jnp.float32),
        pltpu.VMEM((BSZ, HALF), jnp.float32),
        pltpu.VMEM_SHARED((NPAD, HALF), jnp.float32),
        pltpu.SemaphoreType.DMA,
        pltpu.SemaphoreType.DMA,
        pltpu.SemaphoreType.DMA,
        pltpu.SemaphoreType.DMA,
    ],
)
def _agg(y_hbm, src_hbm, dst_hbm, out_hbm, src_v, dst_v, b0, b1, b2, b3,
         acc, g0, g1, g2, g3):
    c = lax.axis_index("c")
    s = lax.axis_index("s")
    bufs = (b0, b1, b2, b3)
    sems = (g0, g1, g2, g3)
    zf = jnp.zeros((16,), jnp.float32)

    def zrow(i, carry):
        def zsub(k, carry2):
            b0[i, pl.ds(k * 16, 16)] = zf
            return carry2

        return lax.fori_loop(0, HALF // 16, zsub, carry)

    lax.fori_loop(0, BSZ, zrow, 0)
    for m in range(STRIPE // BSZ):
        pltpu.sync_copy(b0, acc.at[pl.ds(s * STRIPE + m * BSZ, BSZ)])
    plsc.subcore_barrier()

    def run_phase():
        for k in range(4):
            pltpu.async_copy(y_hbm.at[src_v.at[k]], bufs[k], sems[k])

        def body(ii, carry):
            for k in range(4):
                j = 4 * ii + k

                @pl.when(j < PHB)
                def _(_j=j, _k=k):
                    pltpu.make_async_copy(y_hbm.at[src_v.at[0]], bufs[_k],
                                          sems[_k]).wait()
                    pltpu.sync_copy(bufs[_k], acc.at[dst_v.at[_j]], add=True)

                    @pl.when(_j + 4 < PHB)
                    def _():
                        pltpu.async_copy(y_hbm.at[src_v.at[_j + 4]],
                                         bufs[_k], sems[_k])

            return carry

        lax.fori_loop(0, (PHB + 3) // 4, body, 0)

    for p in range(NBT // PHB):
        pltpu.sync_copy(src_hbm.at[c, s, pl.ds(p * PHB, PHB)], src_v)
        pltpu.sync_copy(dst_hbm.at[s, pl.ds(p * PHB, PHB)], dst_v)
        run_phase()
    plsc.subcore_barrier()

    @pl.when(s < NS - 1)
    def _():
        pltpu.sync_copy(acc.at[pl.ds(s * STRIPE, STRIPE)],
                        out_hbm.at[c, pl.ds(s * STRIPE, STRIPE)])

    @pl.when(s == NS - 1)
    def _():
        pltpu.sync_copy(acc.at[pl.ds((NS - 1) * STRIPE, N - (NS - 1) * STRIPE)],
                        out_hbm.at[c, pl.ds((NS - 1) * STRIPE, N - (NS - 1) * STRIPE)])


@functools.partial(
    pl.kernel,
    out_type=(
        jax.ShapeDtypeStruct((IPAD, D), jnp.float32),
        jax.ShapeDtypeStruct((16, D), jnp.float32),
        jax.ShapeDtypeStruct((16, 16), jnp.float32),
    ),
    mesh=_mesh,
    compiler_params=pltpu.CompilerParams(needs_layout_passes=False),
    scratch_types=[
        pltpu.VMEM((IPT,), jnp.int32),
        pltpu.VMEM((IPT,), jnp.int32),
        pltpu.VMEM((IPT, D), jnp.float32),
        pltpu.VMEM((16, D), jnp.float32),
        pltpu.VMEM((16, 16), jnp.float32),
        pltpu.VMEM((16, D), jnp.float32),
        pltpu.VMEM((16, 16), jnp.float32),
        pltpu.VMEM_SHARED((NS, 16, D), jnp.float32),
        pltpu.VMEM_SHARED((NS, 16, 16), jnp.float32),
        pltpu.SemaphoreType.DMA,
    ],
)
def _tail(embed_hbm, idx_hbm, lab_hbm, raw_out, sums_out, cnt_out,
          idx_v, lab_v, rows_v, sacc, cacc, tmps, tmpc, ssh, csh, sem):
    c = lax.axis_index("c")
    s = lax.axis_index("s")
    pltpu.sync_copy(idx_hbm.at[s], idx_v)
    pltpu.sync_copy(lab_hbm.at[s], lab_v)
    pltpu.async_copy(embed_hbm.at[idx_v], rows_v, sem).wait()
    zf = jnp.zeros((16,), jnp.float32)
    cols = jnp.arange(16, dtype=jnp.int32)
    ones = jnp.full((16,), 1.0, jnp.float32)

    def zero(i, carry):
        def zsub(k, carry2):
            sacc[i, pl.ds(k * 16, 16)] = zf
            return carry2

        lax.fori_loop(0, D // 16, zsub, 0)
        cacc[i, pl.ds(0, 16)] = zf
        return carry

    lax.fori_loop(0, 16, zero, 0)

    def scat(g, carry):
        lv = lab_v[pl.ds(g * 16, 16)]
        base = g * 16
        for e in range(16):
            lbl = jnp.full((16,), lv[e], jnp.int32)

            def chunk(k, carry2, _r=base + e, _l=lbl):
                plsc.addupdate_scatter(sacc, [_l, cols + k * 16],
                                       rows_v[_r, pl.ds(k * 16, 16)])
                return carry2

            lax.fori_loop(0, D // 16, chunk, 0)
            plsc.addupdate_scatter(cacc, [lbl, cols], ones)
        return carry

    lax.fori_loop(0, IPT // 16, scat, 0)
    pltpu.sync_copy(sacc, ssh.at[s])
    pltpu.sync_copy(cacc, csh.at[s])
    plsc.subcore_barrier()

    @pl.when(c == 0)
    def _():
        pltpu.sync_copy(rows_v, raw_out.at[pl.ds(s * IPT, IPT)])

    @pl.when((c == 0) & (s == 0))
    def _():
        def zero2(i, carry):
            def zsub(k, carry2):
                sacc[i, pl.ds(k * 16, 16)] = zf
                return carry2

            lax.fori_loop(0, D // 16, zsub, 0)
            cacc[i, pl.ds(0, 16)] = zf
            return carry

        lax.fori_loop(0, 16, zero2, 0)

        def red(p, carry):
            pltpu.sync_copy(ssh.at[p], tmps)
            pltpu.sync_copy(csh.at[p], tmpc)

            def rrow(i, carry2):
                def rsub(k, carry3):
                    sacc[i, pl.ds(k * 16, 16)] = (
                        sacc[i, pl.ds(k * 16, 16)] + tmps[i, pl.ds(k * 16, 16)])
                    return carry3

                lax.fori_loop(0, D // 16, rsub, 0)
                cacc[i, pl.ds(0, 16)] = (
                    cacc[i, pl.ds(0, 16)] + tmpc[i, pl.ds(0, 16)])
                return carry2

            lax.fori_loop(0, 16, rrow, 0)
            return carry

        lax.fori_loop(0, NS, red, 0)
        pltpu.sync_copy(sacc, sums_out)
        pltpu.sync_copy(cacc, cnt_out)


def _mm_scale(x, w, degp):
    def body(x_ref, w_ref, d_ref, out_ref):
        dinv = lax.rsqrt(d_ref[...])
        z = jnp.dot(x_ref[...], w_ref[...],
                    preferred_element_type=jnp.float32) * dinv
        out_ref[0] = z[:, :HALF]
        out_ref[1] = z[:, HALF:]

    return pl.pallas_call(
        body,
        grid=(N // MMR,),
        in_specs=[
            pl.BlockSpec((MMR, D), lambda i: (i, 0)),
            pl.BlockSpec((D, D), lambda i: (0, 0)),
            pl.BlockSpec((MMR, 1), lambda i: (i, 0)),
        ],
        out_specs=pl.BlockSpec((NC, MMR, HALF), lambda i: (0, i, 0)),
        out_shape=jax.ShapeDtypeStruct((NC, N, HALF), jnp.float32),
    )(x, w, degp)


def _comb_mm(a2, y2, degp, b, w, act=None, emit_comb=False):
    def body(a_ref, y_ref, d_ref, b_ref, w_ref, *outs):
        dinv = lax.rsqrt(d_ref[...])
        z = jnp.concatenate([a_ref[0] + y_ref[0], a_ref[1] + y_ref[1]],
                            axis=1) * dinv + b_ref[...]
        if act == "relu":
            z = jnp.maximum(z, 0.0)
        zn = jnp.dot(z, w_ref[...], preferred_element_type=jnp.float32) * dinv
        outs[0][0] = zn[:, :HALF]
        outs[0][1] = zn[:, HALF:]
        if emit_comb:
            outs[1][...] = z

    rows = lambda i: (i, 0)
    full = lambda i: (0, 0)
    half3 = lambda i: (0, i, 0)
    out_specs = [pl.BlockSpec((NC, MMR, HALF), half3)]
    out_shape = [jax.ShapeDtypeStruct((NC, N, HALF), jnp.float32)]
    if emit_comb:
        out_specs.append(pl.BlockSpec((MMR, D), rows))
        out_shape.append(jax.ShapeDtypeStruct((N, D), jnp.float32))
    out = pl.pallas_call(
        body,
        grid=(N // MMR,),
        in_specs=[
            pl.BlockSpec((NC, MMR, HALF), half3),
            pl.BlockSpec((NC, MMR, HALF), half3),
            pl.BlockSpec((MMR, 1), rows),
            pl.BlockSpec((1, D), full),
            pl.BlockSpec((D, D), full),
        ],
        out_specs=out_specs,
        out_shape=out_shape,
    )(a2, y2, degp, b, w)
    return out if emit_comb else out[0]


def _comb_cond_mm(a2, y2, degp, b, res, wi, bi, wh, bh, wo, bo, origin, w0):
    def body(a_ref, y_ref, d_ref, b_ref, r_ref, wi_r, bi_r, wh_r, bh_r,
             wo_r, bo_r, org_ref, w0_r, out_ref):
        dinv = lax.rsqrt(d_ref[...])
        z = jnp.concatenate([a_ref[0] + y_ref[0], a_ref[1] + y_ref[1]],
                            axis=1) * dinv + b_ref[...] + r_ref[...]
        h = _elu(jnp.dot(z, wi_r[...],
                         preferred_element_type=jnp.float32) + bi_r[...])
        h = _elu(jnp.dot(h, wh_r[...],
                         preferred_element_type=jnp.float32) + bh_r[...])
        xn = (jnp.dot(h, wo_r[...], preferred_element_type=jnp.float32)
              + bo_r[...]) * org_ref[...]
        zn = jnp.dot(xn, w0_r[...], preferred_element_type=jnp.float32) * dinv
        out_ref[0] = zn[:, :HALF]
        out_ref[1] = zn[:, HALF:]

    rows = lambda i: (i, 0)
    full = lambda i: (0, 0)
    half3 = lambda i: (0, i, 0)
    return pl.pallas_call(
        body,
        grid=(N // MMR,),
        in_specs=[
            pl.BlockSpec((NC, MMR, HALF), half3),
            pl.BlockSpec((NC, MMR, HALF), half3),
            pl.BlockSpec((MMR, 1), rows),
            pl.BlockSpec((1, D), full),
            pl.BlockSpec((MMR, D), rows),
            pl.BlockSpec((D, D), full), pl.BlockSpec((1, D), full),
            pl.BlockSpec((D, D), full), pl.BlockSpec((1, D), full),
            pl.BlockSpec((D, D), full), pl.BlockSpec((1, D), full),
            pl.BlockSpec((MMR, D), rows),
            pl.BlockSpec((D, D), full),
        ],
        out_specs=pl.BlockSpec((NC, MMR, HALF), half3),
        out_shape=jax.ShapeDtypeStruct((NC, N, HALF), jnp.float32),
    )(a2, y2, degp, b, res, wi, bi, wh, bh, wo, bo, origin, w0)


def _comb_score(a2, y2, degp, b, wa8, ba8, p8):
    def body(a_ref, y_ref, d_ref, b_ref, wa_r, ba_r, p_r, out_ref):
        dinv = lax.rsqrt(d_ref[...])
        z = jnp.concatenate([a_ref[0] + y_ref[0], a_ref[1] + y_ref[1]],
                            axis=1) * dinv + b_ref[...]
        sc = jnp.dot(z, wa_r[...],
                     preferred_element_type=jnp.float32) + ba_r[...]
        m = jnp.max(sc, axis=1, keepdims=True)
        e = jnp.exp(sc - m)
        w = e / jnp.sum(e, axis=1, keepdims=True)
        out_ref[...] = z + jnp.dot(w, p_r[...],
                                   preferred_element_type=jnp.float32)

    rows = lambda i: (i, 0)
    full = lambda i: (0, 0)
    half3 = lambda i: (0, i, 0)
    return pl.pallas_call(
        body,
        grid=(N // MMR,),
        in_specs=[
            pl.BlockSpec((NC, MMR, HALF), half3),
            pl.BlockSpec((NC, MMR, HALF), half3),
            pl.BlockSpec((MMR, 1), rows),
            pl.BlockSpec((1, D), full),
            pl.BlockSpec((D, 8), full),
            pl.BlockSpec((1, 8), full),
            pl.BlockSpec((8, D), full),
        ],
        out_specs=pl.BlockSpec((MMR, D), rows),
        out_shape=jax.ShapeDtypeStruct((N, D), jnp.float32),
    )(a2, y2, degp, b, wa8, ba8, p8)


def _final(raw, sums, cnts, tfac):
    def body(raw_ref, s_ref, c_ref, t_ref, out_ref):
        ave = s_ref[...] / jnp.maximum(c_ref[...], 1.0) * t_ref[0, 0]
        rw = raw_ref[...]
        rn = jnp.sqrt(jnp.sum(rw * rw, axis=1, keepdims=True))
        an = jnp.sqrt(jnp.sum(ave * ave, axis=1))
        dots = lax.dot_general(rw, ave, (((1,), (1,)), ((), ())),
                               preferred_element_type=jnp.float32)
        denom = jnp.maximum(rn * an[None, :], 1e-8)
        r = dots / denom
        m = jnp.max(r, axis=1, keepdims=True)
        e = jnp.exp(r - m)
        out_ref[...] = e / jnp.sum(e, axis=1, keepdims=True)

    return pl.pallas_call(
        body,
        in_specs=[
            pl.BlockSpec((IPAD, D), lambda: (0, 0)),
            pl.BlockSpec((NB, D), lambda: (0, 0)),
            pl.BlockSpec((NB, 1), lambda: (0, 0)),
            pl.BlockSpec((1, 1), lambda: (0, 0)),
        ],
        out_specs=pl.BlockSpec((IPAD, NB), lambda: (0, 0)),
        out_shape=jax.ShapeDtypeStruct((IPAD, NB), jnp.float32),
    )(raw, sums, cnts, tfac)


def kernel(x, params, edge_index, idx, labels, train):
    x = jnp.asarray(x, jnp.float32)
    src = edge_index[0].astype(jnp.int32)
    dst = edge_index[1].astype(jnp.int32)
    pad = EP - E
    src_p = jnp.concatenate([src, jnp.zeros((pad,), jnp.int32)])
    dst_p = jnp.concatenate([dst, jnp.full((pad,), N, jnp.int32)])
    src3 = src_p.reshape(NS, NBT, BSZ)
    srcoff = jnp.stack([src3, src3 + N])
    dst3 = dst_p.reshape(NS, NBT, BSZ)
    dstd = dst_p.reshape(NS, NBATCH, BATCH)

    degp = _deg(dstd)
    degp_col = degp[:N].reshape(N, 1)

    def agg(y2):
        return _agg(y2.reshape(NC * N, HALF), srcoff, dst3)

    Wg, bg = params["gcn_W"], params["gcn_b"]
    b0, b1, b2 = (b.reshape(1, D) for b in bg)
    w2 = params["gcn_weight2"]
    origin = x
    wa8 = jnp.concatenate([params["Wa"], jnp.zeros((D, 3), jnp.float32)], axis=1)
    ba8 = jnp.concatenate(
        [params["ba"], jnp.full((3,), -1e30, jnp.float32)]).reshape(1, 8)
    p8 = jnp.concatenate([params["p_list"], jnp.zeros((3, D), jnp.float32)],
                         axis=0)

    y2 = _mm_scale(x, Wg[0], degp_col)
    for layer in params["cond"]:
        y2b, e1 = _comb_mm(agg(y2), y2, degp_col, b0, Wg[1], emit_comb=True)
        y2 = _comb_cond_mm(agg(y2b), y2b, degp_col, b1, e1,
                           layer["Wi"] * w2, layer["bi"].reshape(1, D),
                           layer["Wh"], layer["bh"].reshape(1, D),
                           layer["Wo"], layer["bo"].reshape(1, D),
                           origin, Wg[0])
    y2 = _comb_mm(agg(y2), y2, degp_col, b0, Wg[1], act="relu")
    y2 = _comb_mm(agg(y2), y2, degp_col, b1, Wg[2], act="relu")
    embed = _comb_score(agg(y2), y2, degp_col, b2, wa8, ba8, p8)

    idx_p = jnp.concatenate(
        [idx.astype(jnp.int32), jnp.zeros((IPAD - NIDX,), jnp.int32)]
    ).reshape(NS, IPT)
    lab_p = jnp.concatenate(
        [labels.astype(jnp.int32), jnp.full((IPAD - NIDX,), NB, jnp.int32)]
    ).reshape(NS, IPT)
    raw, sums, cnts = _tail(embed, idx_p, lab_p)

    tfac = (jnp.asarray(train) == 1).astype(jnp.float32).reshape(1, 1)
    ret = _final(raw, sums[:NB], cnts[:NB, :1], tfac)
    return ret[:NIDX]


# speedup vs baseline: 1.0226x; 1.0226x over previous
import functools

import jax
import jax.numpy as jnp
from jax import lax
from jax.experimental import pallas as pl
from jax.experimental.pallas import tpu as pltpu
from jax.experimental.pallas import tpu_sc as plsc

N = 10000
E = 160000
D = 256
NB = 10
NIDX = 1000
NC, NS = 2, 16
HALF = D // 2
PHB = 40
BATCH = 128
EP = 163840
NBATCH = EP // NS // BATCH
BSZ = 64
NBT = EP // NS // BSZ
PHB = NBT // 4
NPAD = 10240
STRIPE = NPAD // NS
IPAD = 1024
IPT = IPAD // NS
MMR = 1000

_mesh = plsc.VectorSubcoreMesh(core_axis_name="c", subcore_axis_name="s")


def _elu(v):
    return jnp.where(v > 0, v, jnp.exp(v) - 1.0)


@functools.partial(
    pl.kernel,
    out_type=jax.ShapeDtypeStruct((NPAD,), jnp.float32),
    mesh=_mesh,
    compiler_params=pltpu.CompilerParams(needs_layout_passes=False),
    scratch_types=[
        pltpu.VMEM((NBATCH, BATCH), jnp.int32),
        pltpu.VMEM((NPAD,), jnp.float32),
        pltpu.VMEM((NS, STRIPE), jnp.float32),
        pltpu.VMEM((STRIPE,), jnp.float32),
        pltpu.VMEM_SHARED((NS, NPAD), jnp.float32),
    ],
)
def _deg(dst_hbm, out_hbm, dst_v, acc_v, red_v, res_v, shared):
    c = lax.axis_index("c")
    s = lax.axis_index("s")
    pltpu.sync_copy(dst_hbm.at[s], dst_v)
    zf = jnp.zeros((16,), jnp.float32)

    def zero(i, carry):
        acc_v[pl.ds(i * 16, 16)] = zf
        return carry

    lax.fori_loop(0, NPAD // 16, zero, 0)
    ones = jnp.full((16,), 1.0, jnp.float32)

    def row(j, carry):
        def sub(k, carry2):
            idx = dst_v[j, pl.ds(k * 16, 16)]
            plsc.addupdate_scatter(acc_v, [idx], ones)
            return carry2

        return lax.fori_loop(0, BATCH // 16, sub, carry)

    lax.fori_loop(0, NBATCH, row, 0)
    pltpu.sync_copy(acc_v, shared.at[s])
    plsc.subcore_barrier()
    pltpu.sync_copy(shared.at[:, pl.ds(s * STRIPE, STRIPE)], red_v)

    def reduce(i, carry):
        t = jnp.full((16,), 1.0, jnp.float32)
        for k in range(NS):
            t = t + red_v[k, pl.ds(i * 16, 16)]
        res_v[pl.ds(i * 16, 16)] = t
        return carry

    lax.fori_loop(0, STRIPE // 16, reduce, 0)

    @pl.when(c == 0)
    def _():
        pltpu.sync_copy(res_v, out_hbm.at[pl.ds(s * STRIPE, STRIPE)])


@functools.partial(
    pl.kernel,
    out_type=jax.ShapeDtypeStruct((NC, N, HALF), jnp.float32),
    mesh=_mesh,
    compiler_params=pltpu.CompilerParams(needs_layout_passes=False),
    scratch_types=[
        pltpu.VMEM((PHB, BATCH), jnp.int32),
        pltpu.VMEM((PHB, BATCH), jnp.int32),
        pltpu.VMEM((BATCH, HALF), jnp.float32),
        pltpu.VMEM((BATCH, HALF), jnp.float32),
        pltpu.VMEM_SHARED((NPAD, HALF), jnp.float32),
        pltpu.SemaphoreType.DMA,
        pltpu.SemaphoreType.DMA,
    ],
)
def _agg(y_hbm, src_hbm, dst_hbm, out_hbm, src_v, dst_v, gb0, gb1, acc,
         sem0, sem1):
    c = lax.axis_index("c")
    s = lax.axis_index("s")
    zf = jnp.zeros((16,), jnp.float32)

    def zrow(i, carry):
        def zsub(k, carry2):
            gb0[i, pl.ds(k * 16, 16)] = zf
            return carry2

        return lax.fori_loop(0, HALF // 16, zsub, carry)

    lax.fori_loop(0, BATCH, zrow, 0)
    for m in range(STRIPE // BATCH):
        pltpu.sync_copy(gb0, acc.at[pl.ds(s * STRIPE + m * BATCH, BATCH)])
    plsc.subcore_barrier()

    def run_phase():
        pltpu.async_copy(y_hbm.at[src_v.at[0]], gb0, sem0)

        def body(i, carry):
            j = 2 * i
            pltpu.async_copy(y_hbm.at[src_v.at[j + 1]], gb1, sem1)
            pltpu.make_async_copy(y_hbm.at[src_v.at[0]], gb0, sem0).wait()
            pltpu.sync_copy(gb0, acc.at[dst_v.at[j]], add=True)

            @pl.when(j + 2 < PHB)
            def _():
                pltpu.async_copy(y_hbm.at[src_v.at[j + 2]], gb0, sem0)

            pltpu.make_async_copy(y_hbm.at[src_v.at[0]], gb1, sem1).wait()
            pltpu.sync_copy(gb1, acc.at[dst_v.at[j + 1]], add=True)
            return carry

        lax.fori_loop(0, PHB // 2, body, 0)

    for p in range(NBATCH // PHB):
        pltpu.sync_copy(src_hbm.at[c, s, pl.ds(p * PHB, PHB)], src_v)
        pltpu.sync_copy(dst_hbm.at[s, pl.ds(p * PHB, PHB)], dst_v)
        run_phase()
    plsc.subcore_barrier()

    @pl.when(s < NS - 1)
    def _():
        pltpu.sync_copy(acc.at[pl.ds(s * STRIPE, STRIPE)],
                        out_hbm.at[c, pl.ds(s * STRIPE, STRIPE)])

    @pl.when(s == NS - 1)
    def _():
        pltpu.sync_copy(acc.at[pl.ds((NS - 1) * STRIPE, N - (NS - 1) * STRIPE)],
                        out_hbm.at[c, pl.ds((NS - 1) * STRIPE, N - (NS - 1) * STRIPE)])


@functools.partial(
    pl.kernel,
    out_type=(
        jax.ShapeDtypeStruct((IPAD, D), jnp.float32),
        jax.ShapeDtypeStruct((16, D), jnp.float32),
        jax.ShapeDtypeStruct((16, 16), jnp.float32),
    ),
    mesh=_mesh,
    compiler_params=pltpu.CompilerParams(needs_layout_passes=False),
    scratch_types=[
        pltpu.VMEM((IPT,), jnp.int32),
        pltpu.VMEM((IPT,), jnp.int32),
        pltpu.VMEM((IPT, D), jnp.float32),
        pltpu.VMEM((16, D), jnp.float32),
        pltpu.VMEM((16, 16), jnp.float32),
        pltpu.VMEM((16, D), jnp.float32),
        pltpu.VMEM((16, 16), jnp.float32),
        pltpu.VMEM_SHARED((NS, 16, D), jnp.float32),
        pltpu.VMEM_SHARED((NS, 16, 16), jnp.float32),
        pltpu.SemaphoreType.DMA,
    ],
)
def _tail(embed_hbm, idx_hbm, lab_hbm, raw_out, sums_out, cnt_out,
          idx_v, lab_v, rows_v, sacc, cacc, tmps, tmpc, ssh, csh, sem):
    c = lax.axis_index("c")
    s = lax.axis_index("s")
    pltpu.sync_copy(idx_hbm.at[s], idx_v)
    pltpu.sync_copy(lab_hbm.at[s], lab_v)
    pltpu.async_copy(embed_hbm.at[idx_v], rows_v, sem).wait()
    zf = jnp.zeros((16,), jnp.float32)
    cols = jnp.arange(16, dtype=jnp.int32)
    ones = jnp.full((16,), 1.0, jnp.float32)

    def zero(i, carry):
        def zsub(k, carry2):
            sacc[i, pl.ds(k * 16, 16)] = zf
            return carry2

        lax.fori_loop(0, D // 16, zsub, 0)
        cacc[i, pl.ds(0, 16)] = zf
        return carry

    lax.fori_loop(0, 16, zero, 0)

    def scat(g, carry):
        lv = lab_v[pl.ds(g * 16, 16)]
        base = g * 16
        for e in range(16):
            lbl = jnp.full((16,), lv[e], jnp.int32)

            def chunk(k, carry2, _r=base + e, _l=lbl):
                plsc.addupdate_scatter(sacc, [_l, cols + k * 16],
                                       rows_v[_r, pl.ds(k * 16, 16)])
                return carry2

            lax.fori_loop(0, D // 16, chunk, 0)
            plsc.addupdate_scatter(cacc, [lbl, cols], ones)
        return carry

    lax.fori_loop(0, IPT // 16, scat, 0)
    pltpu.sync_copy(sacc, ssh.at[s])
    pltpu.sync_copy(cacc, csh.at[s])
    plsc.subcore_barrier()

    @pl.when(c == 0)
    def _():
        pltpu.sync_copy(rows_v, raw_out.at[pl.ds(s * IPT, IPT)])

    @pl.when((c == 0) & (s == 0))
    def _():
        def zero2(i, carry):
            def zsub(k, carry2):
                sacc[i, pl.ds(k * 16, 16)] = zf
                return carry2

            lax.fori_loop(0, D // 16, zsub, 0)
            cacc[i, pl.ds(0, 16)] = zf
            return carry

        lax.fori_loop(0, 16, zero2, 0)

        def red(p, carry):
            pltpu.sync_copy(ssh.at[p], tmps)
            pltpu.sync_copy(csh.at[p], tmpc)

            def rrow(i, carry2):
                def rsub(k, carry3):
                    sacc[i, pl.ds(k * 16, 16)] = (
                        sacc[i, pl.ds(k * 16, 16)] + tmps[i, pl.ds(k * 16, 16)])
                    return carry3

                lax.fori_loop(0, D // 16, rsub, 0)
                cacc[i, pl.ds(0, 16)] = (
                    cacc[i, pl.ds(0, 16)] + tmpc[i, pl.ds(0, 16)])
                return carry2

            lax.fori_loop(0, 16, rrow, 0)
            return carry

        lax.fori_loop(0, NS, red, 0)
        pltpu.sync_copy(sacc, sums_out)
        pltpu.sync_copy(cacc, cnt_out)


def _mm_scale(x, w, degp):
    def body(x_ref, w_ref, d_ref, out_ref):
        dinv = lax.rsqrt(d_ref[...])
        z = jnp.dot(x_ref[...], w_ref[...],
                    preferred_element_type=jnp.float32) * dinv
        out_ref[0] = z[:, :HALF]
        out_ref[1] = z[:, HALF:]

    return pl.pallas_call(
        body,
        grid=(N // MMR,),
        in_specs=[
            pl.BlockSpec((MMR, D), lambda i: (i, 0)),
            pl.BlockSpec((D, D), lambda i: (0, 0)),
            pl.BlockSpec((MMR, 1), lambda i: (i, 0)),
        ],
        out_specs=pl.BlockSpec((NC, MMR, HALF), lambda i: (0, i, 0)),
        out_shape=jax.ShapeDtypeStruct((NC, N, HALF), jnp.float32),
    )(x, w, degp)


def _comb_mm(a2, y2, degp, b, w, act=None, emit_comb=False):
    def body(a_ref, y_ref, d_ref, b_ref, w_ref, *outs):
        dinv = lax.rsqrt(d_ref[...])
        z = jnp.concatenate([a_ref[0] + y_ref[0], a_ref[1] + y_ref[1]],
                            axis=1) * dinv + b_ref[...]
        if act == "relu":
            z = jnp.maximum(z, 0.0)
        zn = jnp.dot(z, w_ref[...], preferred_element_type=jnp.float32) * dinv
        outs[0][0] = zn[:, :HALF]
        outs[0][1] = zn[:, HALF:]
        if emit_comb:
            outs[1][...] = z

    rows = lambda i: (i, 0)
    full = lambda i: (0, 0)
    half3 = lambda i: (0, i, 0)
    out_specs = [pl.BlockSpec((NC, MMR, HALF), half3)]
    out_shape = [jax.ShapeDtypeStruct((NC, N, HALF), jnp.float32)]
    if emit_comb:
        out_specs.append(pl.BlockSpec((MMR, D), rows))
        out_shape.append(jax.ShapeDtypeStruct((N, D), jnp.float32))
    out = pl.pallas_call(
        body,
        grid=(N // MMR,),
        in_specs=[
            pl.BlockSpec((NC, MMR, HALF), half3),
            pl.BlockSpec((NC, MMR, HALF), half3),
            pl.BlockSpec((MMR, 1), rows),
            pl.BlockSpec((1, D), full),
            pl.BlockSpec((D, D), full),
        ],
        out_specs=out_specs,
        out_shape=out_shape,
    )(a2, y2, degp, b, w)
    return out if emit_comb else out[0]


def _comb_cond_mm(a2, y2, degp, b, res, wi, bi, wh, bh, wo, bo, origin, w0):
    def body(a_ref, y_ref, d_ref, b_ref, r_ref, wi_r, bi_r, wh_r, bh_r,
             wo_r, bo_r, org_ref, w0_r, out_ref):
        dinv = lax.rsqrt(d_ref[...])
        z = jnp.concatenate([a_ref[0] + y_ref[0], a_ref[1] + y_ref[1]],
                            axis=1) * dinv + b_ref[...] + r_ref[...]
        h = _elu(jnp.dot(z, wi_r[...],
                         preferred_element_type=jnp.float32) + bi_r[...])
        h = _elu(jnp.dot(h, wh_r[...],
                         preferred_element_type=jnp.float32) + bh_r[...])
        xn = (jnp.dot(h, wo_r[...], preferred_element_type=jnp.float32)
              + bo_r[...]) * org_ref[...]
        zn = jnp.dot(xn, w0_r[...], preferred_element_type=jnp.float32) * dinv
        out_ref[0] = zn[:, :HALF]
        out_ref[1] = zn[:, HALF:]

    rows = lambda i: (i, 0)
    full = lambda i: (0, 0)
    half3 = lambda i: (0, i, 0)
    return pl.pallas_call(
        body,
        grid=(N // MMR,),
        in_specs=[
            pl.BlockSpec((NC, MMR, HALF), half3),
            pl.BlockSpec((NC, MMR, HALF), half3),
            pl.BlockSpec((MMR, 1), rows),
            pl.BlockSpec((1, D), full),
            pl.BlockSpec((MMR, D), rows),
            pl.BlockSpec((D, D), full), pl.BlockSpec((1, D), full),
            pl.BlockSpec((D, D), full), pl.BlockSpec((1, D), full),
            pl.BlockSpec((D, D), full), pl.BlockSpec((1, D), full),
            pl.BlockSpec((MMR, D), rows),
            pl.BlockSpec((D, D), full),
        ],
        out_specs=pl.BlockSpec((NC, MMR, HALF), half3),
        out_shape=jax.ShapeDtypeStruct((NC, N, HALF), jnp.float32),
    )(a2, y2, degp, b, res, wi, bi, wh, bh, wo, bo, origin, w0)


def _comb_score(a2, y2, degp, b, wa8, ba8, p8):
    def body(a_ref, y_ref, d_ref, b_ref, wa_r, ba_r, p_r, out_ref):
        dinv = lax.rsqrt(d_ref[...])
        z = jnp.concatenate([a_ref[0] + y_ref[0], a_ref[1] + y_ref[1]],
                            axis=1) * dinv + b_ref[...]
        sc = jnp.dot(z, wa_r[...],
                     preferred_element_type=jnp.float32) + ba_r[...]
        m = jnp.max(sc, axis=1, keepdims=True)
        e = jnp.exp(sc - m)
        w = e / jnp.sum(e, axis=1, keepdims=True)
        out_ref[...] = z + jnp.dot(w, p_r[...],
                                   preferred_element_type=jnp.float32)

    rows = lambda i: (i, 0)
    full = lambda i: (0, 0)
    half3 = lambda i: (0, i, 0)
    return pl.pallas_call(
        body,
        grid=(N // MMR,),
        in_specs=[
            pl.BlockSpec((NC, MMR, HALF), half3),
            pl.BlockSpec((NC, MMR, HALF), half3),
            pl.BlockSpec((MMR, 1), rows),
            pl.BlockSpec((1, D), full),
            pl.BlockSpec((D, 8), full),
            pl.BlockSpec((1, 8), full),
            pl.BlockSpec((8, D), full),
        ],
        out_specs=pl.BlockSpec((MMR, D), rows),
        out_shape=jax.ShapeDtypeStruct((N, D), jnp.float32),
    )(a2, y2, degp, b, wa8, ba8, p8)


def _final(raw, sums, cnts, tfac):
    def body(raw_ref, s_ref, c_ref, t_ref, out_ref):
        ave = s_ref[...] / jnp.maximum(c_ref[...], 1.0) * t_ref[0, 0]
        rw = raw_ref[...]
        rn = jnp.sqrt(jnp.sum(rw * rw, axis=1, keepdims=True))
        an = jnp.sqrt(jnp.sum(ave * ave, axis=1))
        dots = lax.dot_general(rw, ave, (((1,), (1,)), ((), ())),
                               preferred_element_type=jnp.float32)
        denom = jnp.maximum(rn * an[None, :], 1e-8)
        r = dots / denom
        m = jnp.max(r, axis=1, keepdims=True)
        e = jnp.exp(r - m)
        out_ref[...] = e / jnp.sum(e, axis=1, keepdims=True)

    return pl.pallas_call(
        body,
        in_specs=[
            pl.BlockSpec((IPAD, D), lambda: (0, 0)),
            pl.BlockSpec((NB, D), lambda: (0, 0)),
            pl.BlockSpec((NB, 1), lambda: (0, 0)),
            pl.BlockSpec((1, 1), lambda: (0, 0)),
        ],
        out_specs=pl.BlockSpec((IPAD, NB), lambda: (0, 0)),
        out_shape=jax.ShapeDtypeStruct((IPAD, NB), jnp.float32),
    )(raw, sums, cnts, tfac)


def kernel(x, params, edge_index, idx, labels, train):
    x = jnp.asarray(x, jnp.float32)
    src = edge_index[0].astype(jnp.int32)
    dst = edge_index[1].astype(jnp.int32)
    pad = EP - E
    src_p = jnp.concatenate([src, jnp.zeros((pad,), jnp.int32)])
    dst_p = jnp.concatenate([dst, jnp.full((pad,), N, jnp.int32)])
    src3 = src_p.reshape(NS, NBATCH, BATCH)
    srcoff = jnp.stack([src3, src3 + N])
    dst3 = dst_p.reshape(NS, NBATCH, BATCH)

    degp = _deg(dst3)
    degp_col = degp[:N].reshape(N, 1)

    def agg(y2):
        return _agg(y2.reshape(NC * N, HALF), srcoff, dst3)

    Wg, bg = params["gcn_W"], params["gcn_b"]
    b0, b1, b2 = (b.reshape(1, D) for b in bg)
    w2 = params["gcn_weight2"]
    origin = x
    wa8 = jnp.concatenate([params["Wa"], jnp.zeros((D, 3), jnp.float32)], axis=1)
    ba8 = jnp.concatenate(
        [params["ba"], jnp.full((3,), -1e30, jnp.float32)]).reshape(1, 8)
    p8 = jnp.concatenate([params["p_list"], jnp.zeros((3, D), jnp.float32)],
                         axis=0)

    y2 = _mm_scale(x, Wg[0], degp_col)
    for layer in params["cond"]:
        y2b, e1 = _comb_mm(agg(y2), y2, degp_col, b0, Wg[1], emit_comb=True)
        y2 = _comb_cond_mm(agg(y2b), y2b, degp_col, b1, e1,
                           layer["Wi"] * w2, layer["bi"].reshape(1, D),
                           layer["Wh"], layer["bh"].reshape(1, D),
                           layer["Wo"], layer["bo"].reshape(1, D),
                           origin, Wg[0])
    y2 = _comb_mm(agg(y2), y2, degp_col, b0, Wg[1], act="relu")
    y2 = _comb_mm(agg(y2), y2, degp_col, b1, Wg[2], act="relu")
    embed = _comb_score(agg(y2), y2, degp_col, b2, wa8, ba8, p8)

    idx_p = jnp.concatenate(
        [idx.astype(jnp.int32), jnp.zeros((IPAD - NIDX,), jnp.int32)]
    ).reshape(NS, IPT)
    lab_p = jnp.concatenate(
        [labels.astype(jnp.int32), jnp.full((IPAD - NIDX,), NB, jnp.int32)]
    ).reshape(NS, IPT)
    raw, sums, cnts = _tail(embed, idx_p, lab_p)

    tfac = (jnp.asarray(train) == 1).astype(jnp.float32).reshape(1, 1)
    ret = _final(raw, sums[:NB], cnts[:NB, :1], tfac)
    return ret[:NIDX]


# speedup vs baseline: 1.0250x; 1.0024x over previous
import functools

import jax
import jax.numpy as jnp
from jax import lax
from jax.experimental import pallas as pl
from jax.experimental.pallas import tpu as pltpu
from jax.experimental.pallas import tpu_sc as plsc

N = 10000
E = 160000
D = 256
NB = 10
NIDX = 1000
NC, NS = 2, 16
HALF = D // 2
PHB = 40
BATCH = 128
EP = 163840
NBATCH = EP // NS // BATCH
BSZ = 64
NBT = EP // NS // BSZ
PHB = NBT // 4
NPAD = 10240
STRIPE = NPAD // NS
IPAD = 1024
IPT = IPAD // NS
MMR = 1000

_mesh = plsc.VectorSubcoreMesh(core_axis_name="c", subcore_axis_name="s")


def _elu(v):
    return jnp.where(v > 0, v, jnp.exp(v) - 1.0)


@functools.partial(
    pl.kernel,
    out_type=jax.ShapeDtypeStruct((NPAD,), jnp.float32),
    mesh=_mesh,
    compiler_params=pltpu.CompilerParams(needs_layout_passes=False),
    scratch_types=[
        pltpu.VMEM((NBATCH, BATCH), jnp.int32),
        pltpu.VMEM((NPAD,), jnp.float32),
        pltpu.VMEM((NS, STRIPE), jnp.float32),
        pltpu.VMEM((STRIPE,), jnp.float32),
        pltpu.VMEM_SHARED((NS, NPAD), jnp.float32),
    ],
)
def _deg(dst_hbm, out_hbm, dst_v, acc_v, red_v, res_v, shared):
    c = lax.axis_index("c")
    s = lax.axis_index("s")
    pltpu.sync_copy(dst_hbm.at[s], dst_v)
    zf = jnp.zeros((16,), jnp.float32)

    def zero(i, carry):
        acc_v[pl.ds(i * 16, 16)] = zf
        return carry

    lax.fori_loop(0, NPAD // 16, zero, 0)
    ones = jnp.full((16,), 1.0, jnp.float32)

    def row(j, carry):
        def sub(k, carry2):
            idx = dst_v[j, pl.ds(k * 16, 16)]
            plsc.addupdate_scatter(acc_v, [idx], ones)
            return carry2

        return lax.fori_loop(0, BATCH // 16, sub, carry)

    lax.fori_loop(0, NBATCH, row, 0)
    pltpu.sync_copy(acc_v, shared.at[s])
    plsc.subcore_barrier()
    pltpu.sync_copy(shared.at[:, pl.ds(s * STRIPE, STRIPE)], red_v)

    def reduce(i, carry):
        t = jnp.full((16,), 1.0, jnp.float32)
        for k in range(NS):
            t = t + red_v[k, pl.ds(i * 16, 16)]
        res_v[pl.ds(i * 16, 16)] = t
        return carry

    lax.fori_loop(0, STRIPE // 16, reduce, 0)

    @pl.when(c == 0)
    def _():
        pltpu.sync_copy(res_v, out_hbm.at[pl.ds(s * STRIPE, STRIPE)])


@functools.partial(
    pl.kernel,
    out_type=jax.ShapeDtypeStruct((NC, N, HALF), jnp.float32),
    mesh=_mesh,
    compiler_params=pltpu.CompilerParams(needs_layout_passes=False),
    scratch_types=[
        pltpu.VMEM((PHB, BATCH), jnp.int32),
        pltpu.VMEM((PHB, BATCH), jnp.int32),
        pltpu.VMEM((BATCH, HALF), jnp.float32),
        pltpu.VMEM((BATCH, HALF), jnp.float32),
        pltpu.VMEM_SHARED((NPAD, HALF), jnp.float32),
        pltpu.SemaphoreType.DMA,
        pltpu.SemaphoreType.DMA,
    ],
)
def _agg(y_hbm, src_hbm, dst_hbm, out_hbm, src_v, dst_v, gb0, gb1, acc,
         sem0, sem1):
    c = lax.axis_index("c")
    s = lax.axis_index("s")
    zf = jnp.zeros((16,), jnp.float32)

    def zrow(i, carry):
        def zsub(k, carry2):
            gb0[i, pl.ds(k * 16, 16)] = zf
            return carry2

        return lax.fori_loop(0, HALF // 16, zsub, carry)

    lax.fori_loop(0, BATCH, zrow, 0)
    for m in range(STRIPE // BATCH):
        pltpu.sync_copy(gb0, acc.at[pl.ds(s * STRIPE + m * BATCH, BATCH)])
    plsc.subcore_barrier()

    def run_phase():
        pltpu.async_copy(y_hbm.at[src_v.at[0]], gb0, sem0)

        def body(i, carry):
            j = 2 * i
            pltpu.async_copy(y_hbm.at[src_v.at[j + 1]], gb1, sem1)
            pltpu.make_async_copy(y_hbm.at[src_v.at[0]], gb0, sem0).wait()
            pltpu.sync_copy(gb0, acc.at[dst_v.at[j]], add=True)

            @pl.when(j + 2 < PHB)
            def _():
                pltpu.async_copy(y_hbm.at[src_v.at[j + 2]], gb0, sem0)

            pltpu.make_async_copy(y_hbm.at[src_v.at[0]], gb1, sem1).wait()
            pltpu.sync_copy(gb1, acc.at[dst_v.at[j + 1]], add=True)
            return carry

        lax.fori_loop(0, PHB // 2, body, 0)

    for p in range(NBATCH // PHB):
        pltpu.sync_copy(src_hbm.at[c, s, pl.ds(p * PHB, PHB)], src_v)
        pltpu.sync_copy(dst_hbm.at[s, pl.ds(p * PHB, PHB)], dst_v)
        run_phase()
    plsc.subcore_barrier()

    @pl.when(s < NS - 1)
    def _():
        pltpu.sync_copy(acc.at[pl.ds(s * STRIPE, STRIPE)],
                        out_hbm.at[c, pl.ds(s * STRIPE, STRIPE)])

    @pl.when(s == NS - 1)
    def _():
        pltpu.sync_copy(acc.at[pl.ds((NS - 1) * STRIPE, N - (NS - 1) * STRIPE)],
                        out_hbm.at[c, pl.ds((NS - 1) * STRIPE, N - (NS - 1) * STRIPE)])


@functools.partial(
    pl.kernel,
    out_type=(
        jax.ShapeDtypeStruct((IPAD, D), jnp.float32),
        jax.ShapeDtypeStruct((16, D), jnp.float32),
        jax.ShapeDtypeStruct((16, 16), jnp.float32),
    ),
    mesh=_mesh,
    compiler_params=pltpu.CompilerParams(needs_layout_passes=False),
    scratch_types=[
        pltpu.VMEM((IPT,), jnp.int32),
        pltpu.VMEM((IPT,), jnp.int32),
        pltpu.VMEM((IPT, D), jnp.float32),
        pltpu.VMEM((16, D), jnp.float32),
        pltpu.VMEM((16, 16), jnp.float32),
        pltpu.VMEM((16, D), jnp.float32),
        pltpu.VMEM((16, 16), jnp.float32),
        pltpu.VMEM_SHARED((NS, 16, D), jnp.float32),
        pltpu.VMEM_SHARED((NS, 16, 16), jnp.float32),
        pltpu.SemaphoreType.DMA,
    ],
)
def _tail(embed_hbm, idx_hbm, lab_hbm, raw_out, sums_out, cnt_out,
          idx_v, lab_v, rows_v, sacc, cacc, tmps, tmpc, ssh, csh, sem):
    c = lax.axis_index("c")
    s = lax.axis_index("s")
    pltpu.sync_copy(idx_hbm.at[s], idx_v)
    pltpu.sync_copy(lab_hbm.at[s], lab_v)
    pltpu.async_copy(embed_hbm.at[idx_v], rows_v, sem).wait()
    zf = jnp.zeros((16,), jnp.float32)
    cols = jnp.arange(16, dtype=jnp.int32)
    ones = jnp.full((16,), 1.0, jnp.float32)

    def zero(i, carry):
        def zsub(k, carry2):
            sacc[i, pl.ds(k * 16, 16)] = zf
            return carry2

        lax.fori_loop(0, D // 16, zsub, 0)
        cacc[i, pl.ds(0, 16)] = zf
        return carry

    lax.fori_loop(0, 16, zero, 0)

    def scat(g, carry):
        lv = lab_v[pl.ds(g * 16, 16)]
        base = g * 16
        for e in range(16):
            lbl = jnp.full((16,), lv[e], jnp.int32)

            def chunk(k, carry2, _r=base + e, _l=lbl):
                plsc.addupdate_scatter(sacc, [_l, cols + k * 16],
                                       rows_v[_r, pl.ds(k * 16, 16)])
                return carry2

            lax.fori_loop(0, D // 16, chunk, 0)
            plsc.addupdate_scatter(cacc, [lbl, cols], ones)
        return carry

    lax.fori_loop(0, IPT // 16, scat, 0)
    pltpu.sync_copy(sacc, ssh.at[s])
    pltpu.sync_copy(cacc, csh.at[s])
    plsc.subcore_barrier()

    @pl.when(c == 0)
    def _():
        pltpu.sync_copy(rows_v, raw_out.at[pl.ds(s * IPT, IPT)])

    @pl.when((c == 0) & (s == 0))
    def _():
        def zero2(i, carry):
            def zsub(k, carry2):
                sacc[i, pl.ds(k * 16, 16)] = zf
                return carry2

            lax.fori_loop(0, D // 16, zsub, 0)
            cacc[i, pl.ds(0, 16)] = zf
            return carry

        lax.fori_loop(0, 16, zero2, 0)

        def red(p, carry):
            pltpu.sync_copy(ssh.at[p], tmps)
            pltpu.sync_copy(csh.at[p], tmpc)

            def rrow(i, carry2):
                def rsub(k, carry3):
                    sacc[i, pl.ds(k * 16, 16)] = (
                        sacc[i, pl.ds(k * 16, 16)] + tmps[i, pl.ds(k * 16, 16)])
                    return carry3

                lax.fori_loop(0, D // 16, rsub, 0)
                cacc[i, pl.ds(0, 16)] = (
                    cacc[i, pl.ds(0, 16)] + tmpc[i, pl.ds(0, 16)])
                return carry2

            lax.fori_loop(0, 16, rrow, 0)
            return carry

        lax.fori_loop(0, NS, red, 0)
        pltpu.sync_copy(sacc, sums_out)
        pltpu.sync_copy(cacc, cnt_out)


def _mm_scale(x, w, degp):
    def body(x_ref, w_ref, d_ref, out_ref):
        dinv = lax.rsqrt(d_ref[...])
        z = jnp.dot(x_ref[...], w_ref[...],
                    preferred_element_type=jnp.float32) * dinv
        out_ref[0] = z[:, :HALF]
        out_ref[1] = z[:, HALF:]

    return pl.pallas_call(
        body,
        grid=(N // MMR,),
        in_specs=[
            pl.BlockSpec((MMR, D), lambda i: (i, 0)),
            pl.BlockSpec((D, D), lambda i: (0, 0)),
            pl.BlockSpec((MMR, 1), lambda i: (i, 0)),
        ],
        out_specs=pl.BlockSpec((NC, MMR, HALF), lambda i: (0, i, 0)),
        out_shape=jax.ShapeDtypeStruct((NC, N, HALF), jnp.float32),
    )(x, w, degp)


def _comb_mm(a2, y2, degp, b, w, act=None, emit_comb=False):
    def body(a_ref, y_ref, d_ref, b_ref, w_ref, *outs):
        dinv = lax.rsqrt(d_ref[...])
        z = jnp.concatenate([a_ref[0] + y_ref[0], a_ref[1] + y_ref[1]],
                            axis=1) * dinv + b_ref[...]
        if act == "relu":
            z = jnp.maximum(z, 0.0)
        zn = jnp.dot(z, w_ref[...], preferred_element_type=jnp.float32) * dinv
        outs[0][0] = zn[:, :HALF]
        outs[0][1] = zn[:, HALF:]
        if emit_comb:
            outs[1][...] = z

    rows = lambda i: (i, 0)
    full = lambda i: (0, 0)
    half3 = lambda i: (0, i, 0)
    out_specs = [pl.BlockSpec((NC, MMR, HALF), half3)]
    out_shape = [jax.ShapeDtypeStruct((NC, N, HALF), jnp.float32)]
    if emit_comb:
        out_specs.append(pl.BlockSpec((MMR, D), rows))
        out_shape.append(jax.ShapeDtypeStruct((N, D), jnp.float32))
    out = pl.pallas_call(
        body,
        grid=(N // MMR,),
        in_specs=[
            pl.BlockSpec((NC, MMR, HALF), half3),
            pl.BlockSpec((NC, MMR, HALF), half3),
            pl.BlockSpec((MMR, 1), rows),
            pl.BlockSpec((1, D), full),
            pl.BlockSpec((D, D), full),
        ],
        out_specs=out_specs,
        out_shape=out_shape,
    )(a2, y2, degp, b, w)
    return out if emit_comb else out[0]


def _comb_cond_mm(a2, y2, degp, b, res, wi, bi, wh, bh, wo, bo, origin, w0):
    def body(a_ref, y_ref, d_ref, b_ref, r_ref, wi_r, bi_r, wh_r, bh_r,
             wo_r, bo_r, org_ref, w0_r, out_ref):
        dinv = lax.rsqrt(d_ref[...])
        z = jnp.concatenate([a_ref[0] + y_ref[0], a_ref[1] + y_ref[1]],
                            axis=1) * dinv + b_ref[...] + r_ref[...]
        h = _elu(jnp.dot(z, wi_r[...],
                         preferred_element_type=jnp.float32) + bi_r[...])
        h = _elu(jnp.dot(h, wh_r[...],
                         preferred_element_type=jnp.float32) + bh_r[...])
        xn = (jnp.dot(h, wo_r[...], preferred_element_type=jnp.float32)
              + bo_r[...]) * org_ref[...]
        zn = jnp.dot(xn, w0_r[...], preferred_element_type=jnp.float32) * dinv
        out_ref[0] = zn[:, :HALF]
        out_ref[1] = zn[:, HALF:]

    rows = lambda i: (i, 0)
    full = lambda i: (0, 0)
    half3 = lambda i: (0, i, 0)
    return pl.pallas_call(
        body,
        grid=(N // MMR,),
        in_specs=[
            pl.BlockSpec((NC, MMR, HALF), half3),
            pl.BlockSpec((NC, MMR, HALF), half3),
            pl.BlockSpec((MMR, 1), rows),
            pl.BlockSpec((1, D), full),
            pl.BlockSpec((MMR, D), rows),
            pl.BlockSpec((D, D), full), pl.BlockSpec((1, D), full),
            pl.BlockSpec((D, D), full), pl.BlockSpec((1, D), full),
            pl.BlockSpec((D, D), full), pl.BlockSpec((1, D), full),
            pl.BlockSpec((MMR, D), rows),
            pl.BlockSpec((D, D), full),
        ],
        out_specs=pl.BlockSpec((NC, MMR, HALF), half3),
        out_shape=jax.ShapeDtypeStruct((NC, N, HALF), jnp.float32),
    )(a2, y2, degp, b, res, wi, bi, wh, bh, wo, bo, origin, w0)


def _comb_score(a2, y2, degp, b, wa8, ba8, p8):
    def body(a_ref, y_ref, d_ref, b_ref, wa_r, ba_r, p_r, out_ref):
        dinv = lax.rsqrt(d_ref[...])
        z = jnp.concatenate([a_ref[0] + y_ref[0], a_ref[1] + y_ref[1]],
                            axis=1) * dinv + b_ref[...]
        sc = jnp.dot(z, wa_r[...],
                     preferred_element_type=jnp.float32) + ba_r[...]
        m = jnp.max(sc, axis=1, keepdims=True)
        e = jnp.exp(sc - m)
        w = e / jnp.sum(e, axis=1, keepdims=True)
        out_ref[...] = z + jnp.dot(w, p_r[...],
                                   preferred_element_type=jnp.float32)

    rows = lambda i: (i, 0)
    full = lambda i: (0, 0)
    half3 = lambda i: (0, i, 0)
    return pl.pallas_call(
        body,
        grid=(N // MMR,),
        in_specs=[
            pl.BlockSpec((NC, MMR, HALF), half3),
            pl.BlockSpec((NC, MMR, HALF), half3),
            pl.BlockSpec((MMR, 1), rows),
            pl.BlockSpec((1, D), full),
            pl.BlockSpec((D, 8), full),
            pl.BlockSpec((1, 8), full),
            pl.BlockSpec((8, D), full),
        ],
        out_specs=pl.BlockSpec((MMR, D), rows),
        out_shape=jax.ShapeDtypeStruct((N, D), jnp.float32),
    )(a2, y2, degp, b, wa8, ba8, p8)


def _final(raw, sums, cnts, tfac):
    def body(raw_ref, s_ref, c_ref, t_ref, out_ref):
        ave = s_ref[...] / jnp.maximum(c_ref[...], 1.0) * t_ref[0, 0]
        rw = raw_ref[...]
        rn = jnp.sqrt(jnp.sum(rw * rw, axis=1, keepdims=True))
        an = jnp.sqrt(jnp.sum(ave * ave, axis=1))
        dots = lax.dot_general(rw, ave, (((1,), (1,)), ((), ())),
                               preferred_element_type=jnp.float32)
        denom = jnp.maximum(rn * an[None, :], 1e-8)
        r = dots / denom
        m = jnp.max(r, axis=1, keepdims=True)
        e = jnp.exp(r - m)
        out_ref[...] = e / jnp.sum(e, axis=1, keepdims=True)

    return pl.pallas_call(
        body,
        in_specs=[
            pl.BlockSpec((IPAD, D), lambda: (0, 0)),
            pl.BlockSpec((NB, D), lambda: (0, 0)),
            pl.BlockSpec((NB, 1), lambda: (0, 0)),
            pl.BlockSpec((1, 1), lambda: (0, 0)),
        ],
        out_specs=pl.BlockSpec((IPAD, NB), lambda: (0, 0)),
        out_shape=jax.ShapeDtypeStruct((IPAD, NB), jnp.float32),
    )(raw, sums, cnts, tfac)


def kernel(x, params, edge_index, idx, labels, train):
    x = jnp.asarray(x, jnp.float32)
    src = edge_index[0].astype(jnp.int32)
    dst = edge_index[1].astype(jnp.int32)
    pad = EP - E
    src_p = jnp.concatenate([src, jnp.zeros((pad,), jnp.int32)])
    dst_p = jnp.concatenate(
        [dst, N + (jnp.arange(pad, dtype=jnp.int32) % (NPAD - N))])
    src3 = src_p.reshape(NS, NBATCH, BATCH)
    srcoff = jnp.stack([src3, src3 + N])
    dst3 = dst_p.reshape(NS, NBATCH, BATCH)

    degp = _deg(dst3)
    degp_col = degp[:N].reshape(N, 1)

    def agg(y2):
        return _agg(y2.reshape(NC * N, HALF), srcoff, dst3)

    Wg, bg = params["gcn_W"], params["gcn_b"]
    b0, b1, b2 = (b.reshape(1, D) for b in bg)
    w2 = params["gcn_weight2"]
    origin = x
    wa8 = jnp.concatenate([params["Wa"], jnp.zeros((D, 3), jnp.float32)], axis=1)
    ba8 = jnp.concatenate(
        [params["ba"], jnp.full((3,), -1e30, jnp.float32)]).reshape(1, 8)
    p8 = jnp.concatenate([params["p_list"], jnp.zeros((3, D), jnp.float32)],
                         axis=0)

    y2 = _mm_scale(x, Wg[0], degp_col)
    for layer in params["cond"]:
        y2b, e1 = _comb_mm(agg(y2), y2, degp_col, b0, Wg[1], emit_comb=True)
        y2 = _comb_cond_mm(agg(y2b), y2b, degp_col, b1, e1,
                           layer["Wi"] * w2, layer["bi"].reshape(1, D),
                           layer["Wh"], layer["bh"].reshape(1, D),
                           layer["Wo"], layer["bo"].reshape(1, D),
                           origin, Wg[0])
    y2 = _comb_mm(agg(y2), y2, degp_col, b0, Wg[1], act="relu")
    y2 = _comb_mm(agg(y2), y2, degp_col, b1, Wg[2], act="relu")
    embed = _comb_score(agg(y2), y2, degp_col, b2, wa8, ba8, p8)

    idx_p = jnp.concatenate(
        [idx.astype(jnp.int32), jnp.zeros((IPAD - NIDX,), jnp.int32)]
    ).reshape(NS, IPT)
    lab_p = jnp.concatenate(
        [labels.astype(jnp.int32), jnp.full((IPAD - NIDX,), NB, jnp.int32)]
    ).reshape(NS, IPT)
    raw, sums, cnts = _tail(embed, idx_p, lab_p)

    tfac = (jnp.asarray(train) == 1).astype(jnp.float32).reshape(1, 1)
    ret = _final(raw, sums[:NB], cnts[:NB, :1], tfac)
    return ret[:NIDX]


# speedup vs baseline: 1.5101x; 1.4732x over previous
import functools

import jax
import jax.numpy as jnp
from jax import lax
from jax.experimental import pallas as pl
from jax.experimental.pallas import tpu as pltpu
from jax.experimental.pallas import tpu_sc as plsc

N = 10000
E = 160000
D = 256
NB = 10
NIDX = 1000
NC, NS = 2, 16
HALF = D // 2
BATCH = 128
EP = 161792
NBATCH = EP // NS // BATCH
PH0 = 40
PH1 = NBATCH - PH0
NPAD = 10240
STRIPE = NPAD // NS
IPAD = 1024
IPT = IPAD // NS
MMR = 1000

_mesh = plsc.VectorSubcoreMesh(core_axis_name="c", subcore_axis_name="s")


def _elu(v):
    return jnp.where(v > 0, v, jnp.exp(v) - 1.0)


@functools.partial(
    pl.kernel,
    out_type=jax.ShapeDtypeStruct((NPAD,), jnp.float32),
    mesh=_mesh,
    compiler_params=pltpu.CompilerParams(needs_layout_passes=False),
    scratch_types=[
        pltpu.VMEM((NBATCH, BATCH), jnp.int32),
        pltpu.VMEM((NPAD,), jnp.float32),
        pltpu.VMEM((NS, STRIPE), jnp.float32),
        pltpu.VMEM((STRIPE,), jnp.float32),
        pltpu.VMEM_SHARED((NS, NPAD), jnp.float32),
    ],
)
def _deg(dst_hbm, out_hbm, dst_v, acc_v, red_v, res_v, shared):
    c = lax.axis_index("c")
    s = lax.axis_index("s")
    pltpu.sync_copy(dst_hbm.at[s], dst_v)
    zf = jnp.zeros((16,), jnp.float32)

    def zero(i, carry):
        acc_v[pl.ds(i * 16, 16)] = zf
        return carry

    lax.fori_loop(0, NPAD // 16, zero, 0)
    ones = jnp.full((16,), 1.0, jnp.float32)

    def row(j, carry):
        def sub(k, carry2):
            idx = dst_v[j, pl.ds(k * 16, 16)]
            plsc.addupdate_scatter(acc_v, [idx], ones)
            return carry2

        return lax.fori_loop(0, BATCH // 16, sub, carry)

    lax.fori_loop(0, NBATCH, row, 0)
    pltpu.sync_copy(acc_v, shared.at[s])
    plsc.subcore_barrier()
    pltpu.sync_copy(shared.at[:, pl.ds(s * STRIPE, STRIPE)], red_v)

    def reduce(i, carry):
        t = jnp.full((16,), 1.0, jnp.float32)
        for k in range(NS):
            t = t + red_v[k, pl.ds(i * 16, 16)]
        res_v[pl.ds(i * 16, 16)] = t
        return carry

    lax.fori_loop(0, STRIPE // 16, reduce, 0)

    @pl.when(c == 0)
    def _():
        pltpu.sync_copy(res_v, out_hbm.at[pl.ds(s * STRIPE, STRIPE)])


@functools.partial(
    pl.kernel,
    out_type=jax.ShapeDtypeStruct((NC, N, HALF), jnp.float32),
    mesh=_mesh,
    compiler_params=pltpu.CompilerParams(needs_layout_passes=False),
    scratch_types=[
        pltpu.VMEM((PH0, BATCH), jnp.int32),
        pltpu.VMEM((PH0, BATCH), jnp.int32),
        pltpu.VMEM((BATCH, HALF), jnp.float32),
        pltpu.VMEM((BATCH, HALF), jnp.float32),
        pltpu.VMEM_SHARED((NPAD, HALF), jnp.float32),
        pltpu.SemaphoreType.DMA,
        pltpu.SemaphoreType.DMA,
    ],
)
def _agg(y_hbm, src_hbm, dst_hbm, out_hbm, src_v, dst_v, gb0, gb1, acc,
         sem0, sem1):
    c = lax.axis_index("c")
    s = lax.axis_index("s")
    zf = jnp.zeros((16,), jnp.float32)

    def zrow(i, carry):
        def zsub(k, carry2):
            gb0[i, pl.ds(k * 16, 16)] = zf
            return carry2

        return lax.fori_loop(0, HALF // 16, zsub, carry)

    lax.fori_loop(0, BATCH, zrow, 0)
    for m in range(STRIPE // BATCH):
        pltpu.sync_copy(gb0, acc.at[pl.ds(s * STRIPE + m * BATCH, BATCH)])
    plsc.subcore_barrier()

    def run_phase(nb):
        pltpu.async_copy(y_hbm.at[src_v.at[0]], gb0, sem0)

        def body(i, carry):
            j = 2 * i

            @pl.when(j + 1 < nb)
            def _():
                pltpu.async_copy(y_hbm.at[src_v.at[j + 1]], gb1, sem1)

            pltpu.make_async_copy(y_hbm.at[src_v.at[0]], gb0, sem0).wait()
            pltpu.sync_copy(gb0, acc.at[dst_v.at[j]], add=True)

            @pl.when(j + 2 < nb)
            def _():
                pltpu.async_copy(y_hbm.at[src_v.at[j + 2]], gb0, sem0)

            @pl.when(j + 1 < nb)
            def _():
                pltpu.make_async_copy(y_hbm.at[src_v.at[0]], gb1, sem1).wait()
                pltpu.sync_copy(gb1, acc.at[dst_v.at[j + 1]], add=True)

            return carry

        lax.fori_loop(0, (nb + 1) // 2, body, 0)

    pltpu.sync_copy(src_hbm.at[c, s, pl.ds(0, PH0)], src_v)
    pltpu.sync_copy(dst_hbm.at[s, pl.ds(0, PH0)], dst_v)
    run_phase(PH0)
    pltpu.sync_copy(src_hbm.at[c, s, pl.ds(PH0, PH1)], src_v.at[pl.ds(0, PH1)])
    pltpu.sync_copy(dst_hbm.at[s, pl.ds(PH0, PH1)], dst_v.at[pl.ds(0, PH1)])
    run_phase(PH1)
    plsc.subcore_barrier()

    @pl.when(s < NS - 1)
    def _():
        pltpu.sync_copy(acc.at[pl.ds(s * STRIPE, STRIPE)],
                        out_hbm.at[c, pl.ds(s * STRIPE, STRIPE)])

    @pl.when(s == NS - 1)
    def _():
        pltpu.sync_copy(acc.at[pl.ds((NS - 1) * STRIPE, N - (NS - 1) * STRIPE)],
                        out_hbm.at[c, pl.ds((NS - 1) * STRIPE, N - (NS - 1) * STRIPE)])


@functools.partial(
    pl.kernel,
    out_type=(
        jax.ShapeDtypeStruct((IPAD, D), jnp.float32),
        jax.ShapeDtypeStruct((16, D), jnp.float32),
        jax.ShapeDtypeStruct((16, 16), jnp.float32),
    ),
    mesh=_mesh,
    compiler_params=pltpu.CompilerParams(needs_layout_passes=False),
    scratch_types=[
        pltpu.VMEM((IPT,), jnp.int32),
        pltpu.VMEM((IPT,), jnp.int32),
        pltpu.VMEM((IPT, D), jnp.float32),
        pltpu.VMEM((16, D), jnp.float32),
        pltpu.VMEM((16, 16), jnp.float32),
        pltpu.VMEM((16, D), jnp.float32),
        pltpu.VMEM((16, 16), jnp.float32),
        pltpu.VMEM_SHARED((NS, 16, D), jnp.float32),
        pltpu.VMEM_SHARED((NS, 16, 16), jnp.float32),
        pltpu.SemaphoreType.DMA,
    ],
)
def _tail(embed_hbm, idx_hbm, lab_hbm, raw_out, sums_out, cnt_out,
          idx_v, lab_v, rows_v, sacc, cacc, tmps, tmpc, ssh, csh, sem):
    c = lax.axis_index("c")
    s = lax.axis_index("s")
    pltpu.sync_copy(idx_hbm.at[s], idx_v)
    pltpu.sync_copy(lab_hbm.at[s], lab_v)
    pltpu.async_copy(embed_hbm.at[idx_v], rows_v, sem).wait()
    zf = jnp.zeros((16,), jnp.float32)
    cols = jnp.arange(16, dtype=jnp.int32)
    ones = jnp.full((16,), 1.0, jnp.float32)

    def zero(i, carry):
        def zsub(k, carry2):
            sacc[i, pl.ds(k * 16, 16)] = zf
            return carry2

        lax.fori_loop(0, D // 16, zsub, 0)
        cacc[i, pl.ds(0, 16)] = zf
        return carry

    lax.fori_loop(0, 16, zero, 0)

    def scat(g, carry):
        lv = lab_v[pl.ds(g * 16, 16)]
        base = g * 16
        for e in range(16):
            lbl = jnp.full((16,), lv[e], jnp.int32)

            def chunk(k, carry2, _r=base + e, _l=lbl):
                plsc.addupdate_scatter(sacc, [_l, cols + k * 16],
                                       rows_v[_r, pl.ds(k * 16, 16)])
                return carry2

            lax.fori_loop(0, D // 16, chunk, 0)
            plsc.addupdate_scatter(cacc, [lbl, cols], ones)
        return carry

    lax.fori_loop(0, IPT // 16, scat, 0)
    pltpu.sync_copy(sacc, ssh.at[s])
    pltpu.sync_copy(cacc, csh.at[s])
    plsc.subcore_barrier()

    @pl.when(c == 0)
    def _():
        pltpu.sync_copy(rows_v, raw_out.at[pl.ds(s * IPT, IPT)])

    @pl.when((c == 0) & (s == 0))
    def _():
        def zero2(i, carry):
            def zsub(k, carry2):
                sacc[i, pl.ds(k * 16, 16)] = zf
                return carry2

            lax.fori_loop(0, D // 16, zsub, 0)
            cacc[i, pl.ds(0, 16)] = zf
            return carry

        lax.fori_loop(0, 16, zero2, 0)

        def red(p, carry):
            pltpu.sync_copy(ssh.at[p], tmps)
            pltpu.sync_copy(csh.at[p], tmpc)

            def rrow(i, carry2):
                def rsub(k, carry3):
                    sacc[i, pl.ds(k * 16, 16)] = (
                        sacc[i, pl.ds(k * 16, 16)] + tmps[i, pl.ds(k * 16, 16)])
                    return carry3

                lax.fori_loop(0, D // 16, rsub, 0)
                cacc[i, pl.ds(0, 16)] = (
                    cacc[i, pl.ds(0, 16)] + tmpc[i, pl.ds(0, 16)])
                return carry2

            lax.fori_loop(0, 16, rrow, 0)
            return carry

        lax.fori_loop(0, NS, red, 0)
        pltpu.sync_copy(sacc, sums_out)
        pltpu.sync_copy(cacc, cnt_out)


def _mm_scale(x, w, degp):
    def body(x_ref, w_ref, d_ref, out_ref):
        dinv = lax.rsqrt(d_ref[...])
        z = jnp.dot(x_ref[...], w_ref[...],
                    preferred_element_type=jnp.float32) * dinv
        out_ref[0] = z[:, :HALF]
        out_ref[1] = z[:, HALF:]

    return pl.pallas_call(
        body,
        grid=(N // MMR,),
        in_specs=[
            pl.BlockSpec((MMR, D), lambda i: (i, 0)),
            pl.BlockSpec((D, D), lambda i: (0, 0)),
            pl.BlockSpec((MMR, 1), lambda i: (i, 0)),
        ],
        out_specs=pl.BlockSpec((NC, MMR, HALF), lambda i: (0, i, 0)),
        out_shape=jax.ShapeDtypeStruct((NC, N, HALF), jnp.float32),
    )(x, w, degp)


def _comb_mm(a2, y2, degp, b, w, act=None, emit_comb=False):
    def body(a_ref, y_ref, d_ref, b_ref, w_ref, *outs):
        dinv = lax.rsqrt(d_ref[...])
        z = jnp.concatenate([a_ref[0] + y_ref[0], a_ref[1] + y_ref[1]],
                            axis=1) * dinv + b_ref[...]
        if act == "relu":
            z = jnp.maximum(z, 0.0)
        zn = jnp.dot(z, w_ref[...], preferred_element_type=jnp.float32) * dinv
        outs[0][0] = zn[:, :HALF]
        outs[0][1] = zn[:, HALF:]
        if emit_comb:
            outs[1][...] = z

    rows = lambda i: (i, 0)
    full = lambda i: (0, 0)
    half3 = lambda i: (0, i, 0)
    out_specs = [pl.BlockSpec((NC, MMR, HALF), half3)]
    out_shape = [jax.ShapeDtypeStruct((NC, N, HALF), jnp.float32)]
    if emit_comb:
        out_specs.append(pl.BlockSpec((MMR, D), rows))
        out_shape.append(jax.ShapeDtypeStruct((N, D), jnp.float32))
    out = pl.pallas_call(
        body,
        grid=(N // MMR,),
        in_specs=[
            pl.BlockSpec((NC, MMR, HALF), half3),
            pl.BlockSpec((NC, MMR, HALF), half3),
            pl.BlockSpec((MMR, 1), rows),
            pl.BlockSpec((1, D), full),
            pl.BlockSpec((D, D), full),
        ],
        out_specs=out_specs,
        out_shape=out_shape,
    )(a2, y2, degp, b, w)
    return out if emit_comb else out[0]


def _comb_cond_mm(a2, y2, degp, b, res, wi, bi, wh, bh, wo, bo, origin, w0):
    def body(a_ref, y_ref, d_ref, b_ref, r_ref, wi_r, bi_r, wh_r, bh_r,
             wo_r, bo_r, org_ref, w0_r, out_ref):
        dinv = lax.rsqrt(d_ref[...])
        z = jnp.concatenate([a_ref[0] + y_ref[0], a_ref[1] + y_ref[1]],
                            axis=1) * dinv + b_ref[...] + r_ref[...]
        h = _elu(jnp.dot(z, wi_r[...],
                         preferred_element_type=jnp.float32) + bi_r[...])
        h = _elu(jnp.dot(h, wh_r[...],
                         preferred_element_type=jnp.float32) + bh_r[...])
        xn = (jnp.dot(h, wo_r[...], preferred_element_type=jnp.float32)
              + bo_r[...]) * org_ref[...]
        zn = jnp.dot(xn, w0_r[...], preferred_element_type=jnp.float32) * dinv
        out_ref[0] = zn[:, :HALF]
        out_ref[1] = zn[:, HALF:]

    rows = lambda i: (i, 0)
    full = lambda i: (0, 0)
    half3 = lambda i: (0, i, 0)
    return pl.pallas_call(
        body,
        grid=(N // MMR,),
        in_specs=[
            pl.BlockSpec((NC, MMR, HALF), half3),
            pl.BlockSpec((NC, MMR, HALF), half3),
            pl.BlockSpec((MMR, 1), rows),
            pl.BlockSpec((1, D), full),
            pl.BlockSpec((MMR, D), rows),
            pl.BlockSpec((D, D), full), pl.BlockSpec((1, D), full),
            pl.BlockSpec((D, D), full), pl.BlockSpec((1, D), full),
            pl.BlockSpec((D, D), full), pl.BlockSpec((1, D), full),
            pl.BlockSpec((MMR, D), rows),
            pl.BlockSpec((D, D), full),
        ],
        out_specs=pl.BlockSpec((NC, MMR, HALF), half3),
        out_shape=jax.ShapeDtypeStruct((NC, N, HALF), jnp.float32),
    )(a2, y2, degp, b, res, wi, bi, wh, bh, wo, bo, origin, w0)


def _comb_score(a2, y2, degp, b, wa8, ba8, p8):
    def body(a_ref, y_ref, d_ref, b_ref, wa_r, ba_r, p_r, out_ref):
        dinv = lax.rsqrt(d_ref[...])
        z = jnp.concatenate([a_ref[0] + y_ref[0], a_ref[1] + y_ref[1]],
                            axis=1) * dinv + b_ref[...]
        sc = jnp.dot(z, wa_r[...],
                     preferred_element_type=jnp.float32) + ba_r[...]
        m = jnp.max(sc, axis=1, keepdims=True)
        e = jnp.exp(sc - m)
        w = e / jnp.sum(e, axis=1, keepdims=True)
        out_ref[...] = z + jnp.dot(w, p_r[...],
                                   preferred_element_type=jnp.float32)

    rows = lambda i: (i, 0)
    full = lambda i: (0, 0)
    half3 = lambda i: (0, i, 0)
    return pl.pallas_call(
        body,
        grid=(N // MMR,),
        in_specs=[
            pl.BlockSpec((NC, MMR, HALF), half3),
            pl.BlockSpec((NC, MMR, HALF), half3),
            pl.BlockSpec((MMR, 1), rows),
            pl.BlockSpec((1, D), full),
            pl.BlockSpec((D, 8), full),
            pl.BlockSpec((1, 8), full),
            pl.BlockSpec((8, D), full),
        ],
        out_specs=pl.BlockSpec((MMR, D), rows),
        out_shape=jax.ShapeDtypeStruct((N, D), jnp.float32),
    )(a2, y2, degp, b, wa8, ba8, p8)


def _final(raw, sums, cnts, tfac):
    def body(raw_ref, s_ref, c_ref, t_ref, out_ref):
        ave = s_ref[...] / jnp.maximum(c_ref[...], 1.0) * t_ref[0, 0]
        rw = raw_ref[...]
        rn = jnp.sqrt(jnp.sum(rw * rw, axis=1, keepdims=True))
        an = jnp.sqrt(jnp.sum(ave * ave, axis=1))
        dots = lax.dot_general(rw, ave, (((1,), (1,)), ((), ())),
                               preferred_element_type=jnp.float32)
        denom = jnp.maximum(rn * an[None, :], 1e-8)
        r = dots / denom
        m = jnp.max(r, axis=1, keepdims=True)
        e = jnp.exp(r - m)
        out_ref[...] = e / jnp.sum(e, axis=1, keepdims=True)

    return pl.pallas_call(
        body,
        in_specs=[
            pl.BlockSpec((IPAD, D), lambda: (0, 0)),
            pl.BlockSpec((NB, D), lambda: (0, 0)),
            pl.BlockSpec((NB, 1), lambda: (0, 0)),
            pl.BlockSpec((1, 1), lambda: (0, 0)),
        ],
        out_specs=pl.BlockSpec((IPAD, NB), lambda: (0, 0)),
        out_shape=jax.ShapeDtypeStruct((IPAD, NB), jnp.float32),
    )(raw, sums, cnts, tfac)


def kernel(x, params, edge_index, idx, labels, train):
    x = jnp.asarray(x, jnp.float32)
    src = edge_index[0].astype(jnp.int32)
    dst = edge_index[1].astype(jnp.int32)
    pad = EP - E
    src_p = jnp.concatenate([src, jnp.zeros((pad,), jnp.int32)])
    dst_p = jnp.concatenate(
        [dst, N + (jnp.arange(pad, dtype=jnp.int32) % (NPAD - N))])
    src3 = src_p.reshape(NS, NBATCH, BATCH)
    srcoff = jnp.stack([src3, src3 + N])
    dst3 = dst_p.reshape(NS, NBATCH, BATCH)

    degp = _deg(dst3)
    degp_col = degp[:N].reshape(N, 1)

    def agg(y2):
        return _agg(y2.reshape(NC * N, HALF), srcoff, dst3)

    Wg, bg = params["gcn_W"], params["gcn_b"]
    b0, b1, b2 = (b.reshape(1, D) for b in bg)
    w2 = params["gcn_weight2"]
    origin = x
    wa8 = jnp.concatenate([params["Wa"], jnp.zeros((D, 3), jnp.float32)], axis=1)
    ba8 = jnp.concatenate(
        [params["ba"], jnp.full((3,), -1e30, jnp.float32)]).reshape(1, 8)
    p8 = jnp.concatenate([params["p_list"], jnp.zeros((3, D), jnp.float32)],
                         axis=0)

    y2 = _mm_scale(x, Wg[0], degp_col)
    for layer in params["cond"]:
        y2b, e1 = _comb_mm(agg(y2), y2, degp_col, b0, Wg[1], emit_comb=True)
        y2 = _comb_cond_mm(agg(y2b), y2b, degp_col, b1, e1,
                           layer["Wi"] * w2, layer["bi"].reshape(1, D),
                           layer["Wh"], layer["bh"].reshape(1, D),
                           layer["Wo"], layer["bo"].reshape(1, D),
                           origin, Wg[0])
    y2 = _comb_mm(agg(y2), y2, degp_col, b0, Wg[1], act="relu")
    y2 = _comb_mm(agg(y2), y2, degp_col, b1, Wg[2], act="relu")
    embed = _comb_score(agg(y2), y2, degp_col, b2, wa8, ba8, p8)

    idx_p = jnp.concatenate(
        [idx.astype(jnp.int32), jnp.zeros((IPAD - NIDX,), jnp.int32)]
    ).reshape(NS, IPT)
    lab_p = jnp.concatenate(
        [labels.astype(jnp.int32), jnp.full((IPAD - NIDX,), NB, jnp.int32)]
    ).reshape(NS, IPT)
    raw, sums, cnts = _tail(embed, idx_p, lab_p)

    tfac = (jnp.asarray(train) == 1).astype(jnp.float32).reshape(1, 1)
    ret = _final(raw, sums[:NB], cnts[:NB, :1], tfac)
    return ret[:NIDX]


# speedup vs baseline: 1.5235x; 1.0089x over previous
import functools

import jax
import jax.numpy as jnp
from jax import lax
from jax.experimental import pallas as pl
from jax.experimental.pallas import tpu as pltpu
from jax.experimental.pallas import tpu_sc as plsc

N = 10000
E = 160000
D = 256
NB = 10
NIDX = 1000
NC, NS = 2, 16
HALF = D // 2
BATCH = 128
EP = 161792
NBATCH = EP // NS // BATCH
PH0 = 40
PH1 = NBATCH - PH0
NPAD = 10240
STRIPE = NPAD // NS
IPAD = 1024
IPT = IPAD // NS
MMR = 2000

_mesh = plsc.VectorSubcoreMesh(core_axis_name="c", subcore_axis_name="s")


def _elu(v):
    return jnp.where(v > 0, v, jnp.exp(v) - 1.0)


@functools.partial(
    pl.kernel,
    out_type=jax.ShapeDtypeStruct((NPAD,), jnp.float32),
    mesh=_mesh,
    compiler_params=pltpu.CompilerParams(needs_layout_passes=False),
    scratch_types=[
        pltpu.VMEM((NBATCH, BATCH), jnp.int32),
        pltpu.VMEM((NPAD,), jnp.float32),
        pltpu.VMEM((NS, STRIPE), jnp.float32),
        pltpu.VMEM((STRIPE,), jnp.float32),
        pltpu.VMEM_SHARED((NS, NPAD), jnp.float32),
    ],
)
def _deg(dst_hbm, out_hbm, dst_v, acc_v, red_v, res_v, shared):
    c = lax.axis_index("c")
    s = lax.axis_index("s")
    pltpu.sync_copy(dst_hbm.at[s], dst_v)
    zf = jnp.zeros((16,), jnp.float32)

    def zero(i, carry):
        acc_v[pl.ds(i * 16, 16)] = zf
        return carry

    lax.fori_loop(0, NPAD // 16, zero, 0)
    ones = jnp.full((16,), 1.0, jnp.float32)

    def row(j, carry):
        def sub(k, carry2):
            idx = dst_v[j, pl.ds(k * 16, 16)]
            plsc.addupdate_scatter(acc_v, [idx], ones)
            return carry2

        return lax.fori_loop(0, BATCH // 16, sub, carry)

    lax.fori_loop(0, NBATCH, row, 0)
    pltpu.sync_copy(acc_v, shared.at[s])
    plsc.subcore_barrier()
    pltpu.sync_copy(shared.at[:, pl.ds(s * STRIPE, STRIPE)], red_v)

    def reduce(i, carry):
        t = jnp.full((16,), 1.0, jnp.float32)
        for k in range(NS):
            t = t + red_v[k, pl.ds(i * 16, 16)]
        res_v[pl.ds(i * 16, 16)] = t
        return carry

    lax.fori_loop(0, STRIPE // 16, reduce, 0)

    @pl.when(c == 0)
    def _():
        pltpu.sync_copy(res_v, out_hbm.at[pl.ds(s * STRIPE, STRIPE)])


@functools.partial(
    pl.kernel,
    out_type=jax.ShapeDtypeStruct((NC, N, HALF), jnp.float32),
    mesh=_mesh,
    compiler_params=pltpu.CompilerParams(needs_layout_passes=False),
    scratch_types=[
        pltpu.VMEM((PH0, BATCH), jnp.int32),
        pltpu.VMEM((PH0, BATCH), jnp.int32),
        pltpu.VMEM((BATCH, HALF), jnp.float32),
        pltpu.VMEM((BATCH, HALF), jnp.float32),
        pltpu.VMEM_SHARED((NPAD, HALF), jnp.float32),
        pltpu.SemaphoreType.DMA,
        pltpu.SemaphoreType.DMA,
    ],
)
def _agg(y_hbm, src_hbm, dst_hbm, out_hbm, src_v, dst_v, gb0, gb1, acc,
         sem0, sem1):
    c = lax.axis_index("c")
    s = lax.axis_index("s")
    zf = jnp.zeros((16,), jnp.float32)

    def zrow(i, carry):
        def zsub(k, carry2):
            gb0[i, pl.ds(k * 16, 16)] = zf
            return carry2

        return lax.fori_loop(0, HALF // 16, zsub, carry)

    lax.fori_loop(0, BATCH, zrow, 0)
    for m in range(STRIPE // BATCH):
        pltpu.sync_copy(gb0, acc.at[pl.ds(s * STRIPE + m * BATCH, BATCH)])
    plsc.subcore_barrier()

    def run_phase(nb):
        pltpu.async_copy(y_hbm.at[src_v.at[0]], gb0, sem0)

        def body(i, carry):
            j = 2 * i

            @pl.when(j + 1 < nb)
            def _():
                pltpu.async_copy(y_hbm.at[src_v.at[j + 1]], gb1, sem1)

            pltpu.make_async_copy(y_hbm.at[src_v.at[0]], gb0, sem0).wait()
            pltpu.sync_copy(gb0, acc.at[dst_v.at[j]], add=True)

            @pl.when(j + 2 < nb)
            def _():
                pltpu.async_copy(y_hbm.at[src_v.at[j + 2]], gb0, sem0)

            @pl.when(j + 1 < nb)
            def _():
                pltpu.make_async_copy(y_hbm.at[src_v.at[0]], gb1, sem1).wait()
                pltpu.sync_copy(gb1, acc.at[dst_v.at[j + 1]], add=True)

            return carry

        lax.fori_loop(0, (nb + 1) // 2, body, 0)

    pltpu.sync_copy(src_hbm.at[c, s, pl.ds(0, PH0)], src_v)
    pltpu.sync_copy(dst_hbm.at[s, pl.ds(0, PH0)], dst_v)
    run_phase(PH0)
    pltpu.sync_copy(src_hbm.at[c, s, pl.ds(PH0, PH1)], src_v.at[pl.ds(0, PH1)])
    pltpu.sync_copy(dst_hbm.at[s, pl.ds(PH0, PH1)], dst_v.at[pl.ds(0, PH1)])
    run_phase(PH1)
    plsc.subcore_barrier()

    @pl.when(s < NS - 1)
    def _():
        pltpu.sync_copy(acc.at[pl.ds(s * STRIPE, STRIPE)],
                        out_hbm.at[c, pl.ds(s * STRIPE, STRIPE)])

    @pl.when(s == NS - 1)
    def _():
        pltpu.sync_copy(acc.at[pl.ds((NS - 1) * STRIPE, N - (NS - 1) * STRIPE)],
                        out_hbm.at[c, pl.ds((NS - 1) * STRIPE, N - (NS - 1) * STRIPE)])


@functools.partial(
    pl.kernel,
    out_type=(
        jax.ShapeDtypeStruct((IPAD, D), jnp.float32),
        jax.ShapeDtypeStruct((16, D), jnp.float32),
        jax.ShapeDtypeStruct((16, 16), jnp.float32),
    ),
    mesh=_mesh,
    compiler_params=pltpu.CompilerParams(needs_layout_passes=False),
    scratch_types=[
        pltpu.VMEM((IPT,), jnp.int32),
        pltpu.VMEM((IPT,), jnp.int32),
        pltpu.VMEM((IPT, D), jnp.float32),
        pltpu.VMEM((16, D), jnp.float32),
        pltpu.VMEM((16, 16), jnp.float32),
        pltpu.VMEM((16, D), jnp.float32),
        pltpu.VMEM((16, 16), jnp.float32),
        pltpu.VMEM_SHARED((NS, 16, D), jnp.float32),
        pltpu.VMEM_SHARED((NS, 16, 16), jnp.float32),
        pltpu.SemaphoreType.DMA,
    ],
)
def _tail(embed_hbm, idx_hbm, lab_hbm, raw_out, sums_out, cnt_out,
          idx_v, lab_v, rows_v, sacc, cacc, tmps, tmpc, ssh, csh, sem):
    c = lax.axis_index("c")
    s = lax.axis_index("s")
    pltpu.sync_copy(idx_hbm.at[s], idx_v)
    pltpu.sync_copy(lab_hbm.at[s], lab_v)
    pltpu.async_copy(embed_hbm.at[idx_v], rows_v, sem).wait()
    zf = jnp.zeros((16,), jnp.float32)
    cols = jnp.arange(16, dtype=jnp.int32)
    ones = jnp.full((16,), 1.0, jnp.float32)

    def zero(i, carry):
        def zsub(k, carry2):
            sacc[i, pl.ds(k * 16, 16)] = zf
            return carry2

        lax.fori_loop(0, D // 16, zsub, 0)
        cacc[i, pl.ds(0, 16)] = zf
        return carry

    lax.fori_loop(0, 16, zero, 0)

    def scat(g, carry):
        lv = lab_v[pl.ds(g * 16, 16)]
        base = g * 16
        for e in range(16):
            lbl = jnp.full((16,), lv[e], jnp.int32)

            def chunk(k, carry2, _r=base + e, _l=lbl):
                plsc.addupdate_scatter(sacc, [_l, cols + k * 16],
                                       rows_v[_r, pl.ds(k * 16, 16)])
                return carry2

            lax.fori_loop(0, D // 16, chunk, 0)
            plsc.addupdate_scatter(cacc, [lbl, cols], ones)
        return carry

    lax.fori_loop(0, IPT // 16, scat, 0)
    pltpu.sync_copy(sacc, ssh.at[s])
    pltpu.sync_copy(cacc, csh.at[s])
    plsc.subcore_barrier()

    @pl.when(c == 0)
    def _():
        pltpu.sync_copy(rows_v, raw_out.at[pl.ds(s * IPT, IPT)])

    @pl.when((c == 0) & (s == 0))
    def _():
        def zero2(i, carry):
            def zsub(k, carry2):
                sacc[i, pl.ds(k * 16, 16)] = zf
                return carry2

            lax.fori_loop(0, D // 16, zsub, 0)
            cacc[i, pl.ds(0, 16)] = zf
            return carry

        lax.fori_loop(0, 16, zero2, 0)

        def red(p, carry):
            pltpu.sync_copy(ssh.at[p], tmps)
            pltpu.sync_copy(csh.at[p], tmpc)

            def rrow(i, carry2):
                def rsub(k, carry3):
                    sacc[i, pl.ds(k * 16, 16)] = (
                        sacc[i, pl.ds(k * 16, 16)] + tmps[i, pl.ds(k * 16, 16)])
                    return carry3

                lax.fori_loop(0, D // 16, rsub, 0)
                cacc[i, pl.ds(0, 16)] = (
                    cacc[i, pl.ds(0, 16)] + tmpc[i, pl.ds(0, 16)])
                return carry2

            lax.fori_loop(0, 16, rrow, 0)
            return carry

        lax.fori_loop(0, NS, red, 0)
        pltpu.sync_copy(sacc, sums_out)
        pltpu.sync_copy(cacc, cnt_out)


def _mm_scale(x, w, degp):
    def body(x_ref, w_ref, d_ref, out_ref):
        dinv = lax.rsqrt(d_ref[...])
        z = jnp.dot(x_ref[...], w_ref[...],
                    preferred_element_type=jnp.float32) * dinv
        out_ref[0] = z[:, :HALF]
        out_ref[1] = z[:, HALF:]

    return pl.pallas_call(
        body,
        grid=(N // MMR,),
        in_specs=[
            pl.BlockSpec((MMR, D), lambda i: (i, 0)),
            pl.BlockSpec((D, D), lambda i: (0, 0)),
            pl.BlockSpec((MMR, 1), lambda i: (i, 0)),
        ],
        out_specs=pl.BlockSpec((NC, MMR, HALF), lambda i: (0, i, 0)),
        out_shape=jax.ShapeDtypeStruct((NC, N, HALF), jnp.float32),
    )(x, w, degp)


def _comb_mm(a2, y2, degp, b, w, act=None, emit_comb=False):
    def body(a_ref, y_ref, d_ref, b_ref, w_ref, *outs):
        dinv = lax.rsqrt(d_ref[...])
        z = jnp.concatenate([a_ref[0] + y_ref[0], a_ref[1] + y_ref[1]],
                            axis=1) * dinv + b_ref[...]
        if act == "relu":
            z = jnp.maximum(z, 0.0)
        zn = jnp.dot(z, w_ref[...], preferred_element_type=jnp.float32) * dinv
        outs[0][0] = zn[:, :HALF]
        outs[0][1] = zn[:, HALF:]
        if emit_comb:
            outs[1][...] = z

    rows = lambda i: (i, 0)
    full = lambda i: (0, 0)
    half3 = lambda i: (0, i, 0)
    out_specs = [pl.BlockSpec((NC, MMR, HALF), half3)]
    out_shape = [jax.ShapeDtypeStruct((NC, N, HALF), jnp.float32)]
    if emit_comb:
        out_specs.append(pl.BlockSpec((MMR, D), rows))
        out_shape.append(jax.ShapeDtypeStruct((N, D), jnp.float32))
    out = pl.pallas_call(
        body,
        grid=(N // MMR,),
        in_specs=[
            pl.BlockSpec((NC, MMR, HALF), half3),
            pl.BlockSpec((NC, MMR, HALF), half3),
            pl.BlockSpec((MMR, 1), rows),
            pl.BlockSpec((1, D), full),
            pl.BlockSpec((D, D), full),
        ],
        out_specs=out_specs,
        out_shape=out_shape,
    )(a2, y2, degp, b, w)
    return out if emit_comb else out[0]


def _comb_cond_mm(a2, y2, degp, b, res, wi, bi, wh, bh, wo, bo, origin, w0):
    def body(a_ref, y_ref, d_ref, b_ref, r_ref, wi_r, bi_r, wh_r, bh_r,
             wo_r, bo_r, org_ref, w0_r, out_ref):
        dinv = lax.rsqrt(d_ref[...])
        z = jnp.concatenate([a_ref[0] + y_ref[0], a_ref[1] + y_ref[1]],
                            axis=1) * dinv + b_ref[...] + r_ref[...]
        h = _elu(jnp.dot(z, wi_r[...],
                         preferred_element_type=jnp.float32) + bi_r[...])
        h = _elu(jnp.dot(h, wh_r[...],
                         preferred_element_type=jnp.float32) + bh_r[...])
        xn = (jnp.dot(h, wo_r[...], preferred_element_type=jnp.float32)
              + bo_r[...]) * org_ref[...]
        zn = jnp.dot(xn, w0_r[...], preferred_element_type=jnp.float32) * dinv
        out_ref[0] = zn[:, :HALF]
        out_ref[1] = zn[:, HALF:]

    rows = lambda i: (i, 0)
    full = lambda i: (0, 0)
    half3 = lambda i: (0, i, 0)
    return pl.pallas_call(
        body,
        grid=(N // MMR,),
        in_specs=[
            pl.BlockSpec((NC, MMR, HALF), half3),
            pl.BlockSpec((NC, MMR, HALF), half3),
            pl.BlockSpec((MMR, 1), rows),
            pl.BlockSpec((1, D), full),
            pl.BlockSpec((MMR, D), rows),
            pl.BlockSpec((D, D), full), pl.BlockSpec((1, D), full),
            pl.BlockSpec((D, D), full), pl.BlockSpec((1, D), full),
            pl.BlockSpec((D, D), full), pl.BlockSpec((1, D), full),
            pl.BlockSpec((MMR, D), rows),
            pl.BlockSpec((D, D), full),
        ],
        out_specs=pl.BlockSpec((NC, MMR, HALF), half3),
        out_shape=jax.ShapeDtypeStruct((NC, N, HALF), jnp.float32),
    )(a2, y2, degp, b, res, wi, bi, wh, bh, wo, bo, origin, w0)


def _comb_score(a2, y2, degp, b, wa8, ba8, p8):
    def body(a_ref, y_ref, d_ref, b_ref, wa_r, ba_r, p_r, out_ref):
        dinv = lax.rsqrt(d_ref[...])
        z = jnp.concatenate([a_ref[0] + y_ref[0], a_ref[1] + y_ref[1]],
                            axis=1) * dinv + b_ref[...]
        sc = jnp.dot(z, wa_r[...],
                     preferred_element_type=jnp.float32) + ba_r[...]
        m = jnp.max(sc, axis=1, keepdims=True)
        e = jnp.exp(sc - m)
        w = e / jnp.sum(e, axis=1, keepdims=True)
        out_ref[...] = z + jnp.dot(w, p_r[...],
                                   preferred_element_type=jnp.float32)

    rows = lambda i: (i, 0)
    full = lambda i: (0, 0)
    half3 = lambda i: (0, i, 0)
    return pl.pallas_call(
        body,
        grid=(N // MMR,),
        in_specs=[
            pl.BlockSpec((NC, MMR, HALF), half3),
            pl.BlockSpec((NC, MMR, HALF), half3),
            pl.BlockSpec((MMR, 1), rows),
            pl.BlockSpec((1, D), full),
            pl.BlockSpec((D, 8), full),
            pl.BlockSpec((1, 8), full),
            pl.BlockSpec((8, D), full),
        ],
        out_specs=pl.BlockSpec((MMR, D), rows),
        out_shape=jax.ShapeDtypeStruct((N, D), jnp.float32),
    )(a2, y2, degp, b, wa8, ba8, p8)


def _final(raw, sums, cnts, tfac):
    def body(raw_ref, s_ref, c_ref, t_ref, out_ref):
        ave = s_ref[...] / jnp.maximum(c_ref[...], 1.0) * t_ref[0, 0]
        rw = raw_ref[...]
        rn = jnp.sqrt(jnp.sum(rw * rw, axis=1, keepdims=True))
        an = jnp.sqrt(jnp.sum(ave * ave, axis=1))
        dots = lax.dot_general(rw, ave, (((1,), (1,)), ((), ())),
                               preferred_element_type=jnp.float32)
        denom = jnp.maximum(rn * an[None, :], 1e-8)
        r = dots / denom
        m = jnp.max(r, axis=1, keepdims=True)
        e = jnp.exp(r - m)
        out_ref[...] = e / jnp.sum(e, axis=1, keepdims=True)

    return pl.pallas_call(
        body,
        in_specs=[
            pl.BlockSpec((IPAD, D), lambda: (0, 0)),
            pl.BlockSpec((NB, D), lambda: (0, 0)),
            pl.BlockSpec((NB, 1), lambda: (0, 0)),
            pl.BlockSpec((1, 1), lambda: (0, 0)),
        ],
        out_specs=pl.BlockSpec((IPAD, NB), lambda: (0, 0)),
        out_shape=jax.ShapeDtypeStruct((IPAD, NB), jnp.float32),
    )(raw, sums, cnts, tfac)


def kernel(x, params, edge_index, idx, labels, train):
    x = jnp.asarray(x, jnp.float32)
    src = edge_index[0].astype(jnp.int32)
    dst = edge_index[1].astype(jnp.int32)
    pad = EP - E
    src_p = jnp.concatenate([src, jnp.zeros((pad,), jnp.int32)])
    dst_p = jnp.concatenate(
        [dst, N + (jnp.arange(pad, dtype=jnp.int32) % (NPAD - N))])
    src3 = src_p.reshape(NS, NBATCH, BATCH)
    srcoff = jnp.stack([src3, src3 + N])
    dst3 = dst_p.reshape(NS, NBATCH, BATCH)

    degp = _deg(dst3)
    degp_col = degp[:N].reshape(N, 1)

    def agg(y2):
        return _agg(y2.reshape(NC * N, HALF), srcoff, dst3)

    Wg, bg = params["gcn_W"], params["gcn_b"]
    b0, b1, b2 = (b.reshape(1, D) for b in bg)
    w2 = params["gcn_weight2"]
    origin = x
    wa8 = jnp.concatenate([params["Wa"], jnp.zeros((D, 3), jnp.float32)], axis=1)
    ba8 = jnp.concatenate(
        [params["ba"], jnp.full((3,), -1e30, jnp.float32)]).reshape(1, 8)
    p8 = jnp.concatenate([params["p_list"], jnp.zeros((3, D), jnp.float32)],
                         axis=0)

    y2 = _mm_scale(x, Wg[0], degp_col)
    for layer in params["cond"]:
        y2b, e1 = _comb_mm(agg(y2), y2, degp_col, b0, Wg[1], emit_comb=True)
        y2 = _comb_cond_mm(agg(y2b), y2b, degp_col, b1, e1,
                           layer["Wi"] * w2, layer["bi"].reshape(1, D),
                           layer["Wh"], layer["bh"].reshape(1, D),
                           layer["Wo"], layer["bo"].reshape(1, D),
                           origin, Wg[0])
    y2 = _comb_mm(agg(y2), y2, degp_col, b0, Wg[1], act="relu")
    y2 = _comb_mm(agg(y2), y2, degp_col, b1, Wg[2], act="relu")
    embed = _comb_score(agg(y2), y2, degp_col, b2, wa8, ba8, p8)

    idx_p = jnp.concatenate(
        [idx.astype(jnp.int32), jnp.zeros((IPAD - NIDX,), jnp.int32)]
    ).reshape(NS, IPT)
    lab_p = jnp.concatenate(
        [labels.astype(jnp.int32), jnp.full((IPAD - NIDX,), NB, jnp.int32)]
    ).reshape(NS, IPT)
    raw, sums, cnts = _tail(embed, idx_p, lab_p)

    tfac = (jnp.asarray(train) == 1).astype(jnp.float32).reshape(1, 1)
    ret = _final(raw, sums[:NB], cnts[:NB, :1], tfac)
    return ret[:NIDX]
